# Initial kernel scaffold; baseline (speedup 1.0000x reference)
import jax
import jax.numpy as jnp
from jax.experimental import pallas as pl


def kernel(x, edge_index, edge_attr, W1, b1, Wout, bout, root_param, kbias, ps0_W, ps0_b, ps0_r, ps1_W, ps1_b, ps1_r, ps2_W, ps2_b, ps2_r, psout_W, psout_b, psout_r, bn_g, bn_b, dW1, db1, dW2, db2, dW3, db3):
    raise NotImplementedError("write your pallas kernel here")


# speedup vs baseline: 2.5211x; 2.5211x over previous
import functools

import jax
import jax.numpy as jnp
from jax import lax
from jax.experimental import pallas as pl
from jax.experimental.pallas import tpu as pltpu
from jax.experimental.pallas import tpu_sc as plsc

_N = 10000
_E = 160000
_W = 16
_NC = 2
_NS = 16
_NW = _NC * _NS
_CH = 128
_NCHUNK = 40
_EW = _CH * _NCHUNK
_E_PAD = _EW * _NW
_ND = 10240
_STRIPE = _ND // _NS
_DUMMY = _N

_f32 = jnp.float32


def _leaky(x):
    return jnp.where(x >= 0, x, 0.1 * x)


def _ps_scalar(a, w_ref, b_ref, r_ref):
    out = r_ref[0:1, :] * (a * w_ref[0:1, :] + b_ref[0:1, :])
    x1 = _leaky(a * w_ref[1:2, :] + b_ref[1:2, :])
    out += r_ref[1:2, :] * x1
    x2 = _leaky(a * w_ref[2:3, :] + b_ref[2:3, :])
    out += r_ref[2:3, :] * (x2 * x2)
    x3 = _leaky(a * w_ref[3:4, :] + b_ref[3:4, :])
    out += r_ref[3:4, :] * (x3 * x3 * x3)
    return out


def _ps_mat(x, w_ref, b_ref, r_ref):
    out = r_ref[0:1, :] * (
        jnp.dot(x, w_ref[0], preferred_element_type=_f32) + b_ref[0:1, :])
    x1 = _leaky(jnp.dot(x, w_ref[1], preferred_element_type=_f32) + b_ref[1:2, :])
    out += r_ref[1:2, :] * x1
    x2 = _leaky(jnp.dot(x, w_ref[2], preferred_element_type=_f32) + b_ref[2:3, :])
    out += r_ref[2:3, :] * (x2 * x2)
    x3 = _leaky(jnp.dot(x, w_ref[3], preferred_element_type=_f32) + b_ref[3:4, :])
    out += r_ref[3:4, :] * (x3 * x3 * x3)
    return out


def _bn_apply(x, st_ref, g_ref, b_ref):
    return ((x - st_ref[0:1, :]) * lax.rsqrt(st_ref[1:2, :] + 1e-5)
            * g_ref[0:1, :] + b_ref[0:1, :])


def _tc_h0(x, w1, b1):
    b = 2000

    def body(x_ref, w_ref, b_ref, o_ref):
        o_ref[...] = (jnp.dot(x_ref[...], w_ref[...], preferred_element_type=_f32)
                      + b_ref[...])

    return pl.pallas_call(
        body,
        grid=(_N // b,),
        in_specs=[pl.BlockSpec((b, 128), lambda i: (i, 0)),
                  pl.BlockSpec((128, _W), lambda i: (0, 0)),
                  pl.BlockSpec((1, _W), lambda i: (0, 0))],
        out_specs=pl.BlockSpec((b, _W), lambda i: (i, 0)),
        out_shape=jax.ShapeDtypeStruct((_N, _W), _f32),
    )(x, w1, b1)


def _stats_finalize(o_ref, nblk, i):
    @pl.when(i == nblk - 1)
    def _():
        s = o_ref[0:1, :]
        q = o_ref[1:2, :]
        m = s * (1.0 / _E)
        v = q * (1.0 / _E) - m * m
        o_ref[0:1, :] = m
        o_ref[1:2, :] = v


def _tc_stats1(attr, w0, b0, r0, w1, b1, r1):
    b = 1600
    nblk = _E // b

    def body(a_ref, w0r, b0r, r0r, w1r, b1r, r1r, o_ref):
        i = pl.program_id(0)

        @pl.when(i == 0)
        def _():
            o_ref[...] = jnp.zeros((2, 64), _f32)

        kx0 = _ps_scalar(a_ref[...], w0r, b0r, r0r)
        kx1 = _ps_mat(kx0, w1r, b1r, r1r)
        o_ref[0:1, :] += jnp.sum(kx1, axis=0, keepdims=True)
        o_ref[1:2, :] += jnp.sum(kx1 * kx1, axis=0, keepdims=True)
        _stats_finalize(o_ref, nblk, i)

    full = lambda s: pl.BlockSpec(s, lambda i: tuple(0 for _ in s))
    return pl.pallas_call(
        body,
        grid=(nblk,),
        in_specs=[pl.BlockSpec((b, 1), lambda i: (i, 0)),
                  full((4, 64)), full((4, 64)), full((4, 64)),
                  full((4, 64, 64)), full((4, 64)), full((4, 64))],
        out_specs=pl.BlockSpec((2, 64), lambda i: (0, 0)),
        out_shape=jax.ShapeDtypeStruct((2, 64), _f32),
    )(attr, w0, b0, r0, w1, b1, r1)


def _tc_stats2(attr, w0, b0, r0, w1, b1, r1, st1, bng, bnb, w2, b2, r2):
    b = 1600
    nblk = _E // b

    def body(a_ref, w0r, b0r, r0r, w1r, b1r, r1r, st1r, gr, br, w2r, b2r, r2r,
             o_ref):
        i = pl.program_id(0)

        @pl.when(i == 0)
        def _():
            o_ref[...] = jnp.zeros((2, 64), _f32)

        kx0 = _ps_scalar(a_ref[...], w0r, b0r, r0r)
        kx1 = _ps_mat(kx0, w1r, b1r, r1r)
        bn1 = _bn_apply(kx1, st1r, gr, br)
        kx2 = _ps_mat(bn1, w2r, b2r, r2r)
        o_ref[0:1, :] += jnp.sum(kx2, axis=0, keepdims=True)
        o_ref[1:2, :] += jnp.sum(kx2 * kx2, axis=0, keepdims=True)
        _stats_finalize(o_ref, nblk, i)

    full = lambda s: pl.BlockSpec(s, lambda i: tuple(0 for _ in s))
    return pl.pallas_call(
        body,
        grid=(nblk,),
        in_specs=[pl.BlockSpec((b, 1), lambda i: (i, 0)),
                  full((4, 64)), full((4, 64)), full((4, 64)),
                  full((4, 64, 64)), full((4, 64)), full((4, 64)),
                  full((2, 64)), full((1, 64)), full((1, 64)),
                  full((4, 64, 64)), full((4, 64)), full((4, 64))],
        out_specs=pl.BlockSpec((2, 64), lambda i: (0, 0)),
        out_shape=jax.ShapeDtypeStruct((2, 64), _f32),
    )(attr, w0, b0, r0, w1, b1, r1, st1, bng, bnb, w2, b2, r2)


def _tc_weights(attr_pad, w0, b0, r0, w1, b1, r1, st1, bng, bnb,
                w2, b2, r2, st2, wo, bo, ro, dw1, db1, dw2, db2, dw3, db3):
    b = 1024
    nblk = _E_PAD // b

    def body(a_ref, w0r, b0r, r0r, w1r, b1r, r1r, st1r, gr, br,
             w2r, b2r, r2r, st2r, wor, bor, ror, d1r, e1r, d2r, e2r, d3r, e3r,
             wk_ref, ws_ref):
        a = a_ref[...]
        kx0 = _ps_scalar(a, w0r, b0r, r0r)
        kx1 = _ps_mat(kx0, w1r, b1r, r1r)
        bn1 = _bn_apply(kx1, st1r, gr, br)
        kx2 = _ps_mat(bn1, w2r, b2r, r2r)
        bn2 = _bn_apply(kx2, st2r, gr, br)
        wk = _ps_mat(bn2, wor, bor, ror)
        hd = jnp.maximum(a * d1r[...] + e1r[...], 0.0)
        hd = jnp.maximum(jnp.dot(hd, d2r[...], preferred_element_type=_f32)
                         + e2r[...], 0.0)
        wop = jnp.dot(hd, d3r[...], preferred_element_type=_f32) + e3r[...]
        wk_ref[...] = wk
        ws_ref[...] = wk + wop

    full = lambda s: pl.BlockSpec(s, lambda i: tuple(0 for _ in s))
    return pl.pallas_call(
        body,
        grid=(nblk,),
        in_specs=[pl.BlockSpec((b, 1), lambda i: (i, 0)),
                  full((4, 64)), full((4, 64)), full((4, 64)),
                  full((4, 64, 64)), full((4, 64)), full((4, 64)),
                  full((2, 64)), full((1, 64)), full((1, 64)),
                  full((4, 64, 64)), full((4, 64)), full((4, 64)),
                  full((2, 64)),
                  full((4, 64, 256)), full((4, 256)), full((4, 256)),
                  full((1, 128)), full((1, 128)),
                  full((128, 128)), full((1, 128)),
                  full((128, 256)), full((1, 256))],
        out_specs=[pl.BlockSpec((b, 256), lambda i: (i, 0)),
                   pl.BlockSpec((b, 256), lambda i: (i, 0))],
        out_shape=[jax.ShapeDtypeStruct((_E_PAD, 256), _f32),
                   jax.ShapeDtypeStruct((_E_PAD, 256), _f32)],
    )(attr_pad, w0, b0, r0, w1, b1, r1, st1, bng, bnb,
      w2, b2, r2, st2, wo, bo, ro, dw1, db1, dw2, db2, dw3, db3)


def _tc_msg(xj, xi, wk, wsum, rep, red):
    b = 2048
    nblk = _E_PAD // b

    def body(xj_ref, xi_ref, wk_ref, ws_ref, rep_ref, red_ref, o_ref):
        xjr = jnp.dot(xj_ref[...], rep_ref[...], preferred_element_type=_f32)
        xir = jnp.dot(xi_ref[...], rep_ref[...], preferred_element_type=_f32)
        t = xjr * ws_ref[...] - xir * wk_ref[...]
        o_ref[...] = jnp.dot(t, red_ref[...], preferred_element_type=_f32)

    full = lambda s: pl.BlockSpec(s, lambda i: tuple(0 for _ in s))
    return pl.pallas_call(
        body,
        grid=(nblk,),
        in_specs=[pl.BlockSpec((b, _W), lambda i: (i, 0)),
                  pl.BlockSpec((b, _W), lambda i: (i, 0)),
                  pl.BlockSpec((b, 256), lambda i: (i, 0)),
                  pl.BlockSpec((b, 256), lambda i: (i, 0)),
                  full((_W, 256)), full((256, _W))],
        out_specs=pl.BlockSpec((b, _W), lambda i: (i, 0)),
        out_shape=jax.ShapeDtypeStruct((_E_PAD, _W), _f32),
    )(xj, xi, wk, wsum, rep, red)


def _tc_update(p0, p1, c0, c1, h, root, kb):
    b = 2000

    def body(p0r, p1r, c0r, c1r, hr, rtr, kbr, o_ref):
        agg = (p0r[...] + p1r[...]) / jnp.maximum(c0r[...] + c1r[...], 1.0)
        hv = jnp.dot(hr[...], rtr[...], preferred_element_type=_f32)
        o_ref[...] = jnp.maximum(agg + hv + kbr[...], 0.0)

    full = lambda s: pl.BlockSpec(s, lambda i: tuple(0 for _ in s))
    blk = pl.BlockSpec((b, _W), lambda i: (i, 0))
    return pl.pallas_call(
        body,
        grid=(_N // b,),
        in_specs=[blk, blk, blk, blk, blk, full((_W, _W)), full((1, _W))],
        out_specs=pl.BlockSpec((b, _W), lambda i: (i, 0)),
        out_shape=jax.ShapeDtypeStruct((_N, _W), _f32),
    )(p0, p1, c0, c1, h, root, kb)


def _tc_update_out(p0, p1, c0, c1, h, root, kb, wout, bout):
    b = 2000

    def body(p0r, p1r, c0r, c1r, hr, rtr, kbr, wor, bor, o_ref):
        agg = (p0r[...] + p1r[...]) / jnp.maximum(c0r[...] + c1r[...], 1.0)
        hv = jnp.dot(hr[...], rtr[...], preferred_element_type=_f32)
        hn = jnp.maximum(agg + hv + kbr[...], 0.0)
        o_ref[...] = (jnp.dot(hn, wor[...], preferred_element_type=_f32)
                      + bor[...])

    full = lambda s: pl.BlockSpec(s, lambda i: tuple(0 for _ in s))
    blk = pl.BlockSpec((b, _W), lambda i: (i, 0))
    return pl.pallas_call(
        body,
        grid=(_N // b,),
        in_specs=[blk, blk, blk, blk, blk, full((_W, _W)), full((1, _W)),
                  full((_W, 128)), full((1, 128))],
        out_specs=pl.BlockSpec((b, 128), lambda i: (i, 0)),
        out_shape=jax.ShapeDtypeStruct((_N, 128), _f32),
    )(p0, p1, c0, c1, h, root, kb, wout, bout)


@functools.cache
def _sc_kernels():
    mesh = plsc.VectorSubcoreMesh(core_axis_name="c", subcore_axis_name="s")
    params = pltpu.CompilerParams(use_tc_tiling_on_sc=False)

    @functools.partial(
        pl.kernel,
        out_type=[jax.ShapeDtypeStruct((_E_PAD, _W), _f32),
                  jax.ShapeDtypeStruct((_E_PAD, _W), _f32)],
        mesh=mesh,
        compiler_params=params,
        scratch_types=[pltpu.VMEM((_NCHUNK, _CH), jnp.int32),
                       pltpu.VMEM((_EW, _W), _f32),
                       pltpu.SemaphoreType.DMA],
    )
    def sc_gather(h_hbm, src_hbm, dst_hbm, xj_hbm, xi_hbm, idx_v, rows_v,
                  sem):
        w = lax.axis_index("s") * _NC + lax.axis_index("c")
        for idx_hbm, out_hbm in ((src_hbm, xj_hbm), (dst_hbm, xi_hbm)):
            pltpu.sync_copy(idx_hbm.at[w], idx_v)

            def body(j, carry):
                pltpu.async_copy(h_hbm.at[idx_v.at[j]],
                                 rows_v.at[pl.ds(j * _CH, _CH)], sem).wait()
                return carry

            lax.fori_loop(0, _NCHUNK, body, 0)
            pltpu.sync_copy(rows_v, out_hbm.at[pl.ds(w * _EW, _EW)])

    @functools.partial(
        pl.kernel,
        out_type=jax.ShapeDtypeStruct((_NC * _ND, _W), _f32),
        mesh=mesh,
        compiler_params=params,
        scratch_types=[pltpu.VMEM((_NCHUNK, _CH), jnp.int32),
                       pltpu.VMEM((_EW, _W), _f32),
                       pltpu.VMEM_SHARED((_ND, _W), _f32)],
    )
    def sc_scatter(msg_hbm, dst_hbm, zeros_hbm, out_hbm, idx_v, msg_v,
                   agg_sh):
        cid = lax.axis_index("c")
        sid = lax.axis_index("s")
        w = sid * _NC + cid
        pltpu.sync_copy(zeros_hbm.at[pl.ds(sid * _STRIPE, _STRIPE)],
                        agg_sh.at[pl.ds(sid * _STRIPE, _STRIPE)])
        plsc.subcore_barrier()
        pltpu.sync_copy(dst_hbm.at[w], idx_v)
        pltpu.sync_copy(msg_hbm.at[pl.ds(w * _EW, _EW)], msg_v)

        def body(j, carry):
            pltpu.sync_copy(msg_v.at[pl.ds(j * _CH, _CH)],
                            agg_sh.at[idx_v.at[j]], add=True)
            return carry

        lax.fori_loop(0, _NCHUNK, body, 0)
        plsc.subcore_barrier()
        pltpu.sync_copy(agg_sh.at[pl.ds(sid * _STRIPE, _STRIPE)],
                        out_hbm.at[pl.ds(cid * _ND + sid * _STRIPE, _STRIPE)])

    @functools.partial(
        pl.kernel,
        out_type=jax.ShapeDtypeStruct((_NC * _ND, _W), _f32),
        mesh=mesh,
        compiler_params=params,
        scratch_types=[pltpu.VMEM((_NCHUNK, _CH), jnp.int32),
                       pltpu.VMEM((_CH, _W), _f32),
                       pltpu.VMEM_SHARED((_ND, _W), _f32)],
    )
    def sc_count(dst_hbm, zeros_hbm, ones_hbm, out_hbm, idx_v, ones_v,
                 cnt_sh):
        cid = lax.axis_index("c")
        sid = lax.axis_index("s")
        w = sid * _NC + cid
        pltpu.sync_copy(zeros_hbm.at[pl.ds(sid * _STRIPE, _STRIPE)],
                        cnt_sh.at[pl.ds(sid * _STRIPE, _STRIPE)])
        pltpu.sync_copy(ones_hbm, ones_v)
        plsc.subcore_barrier()
        pltpu.sync_copy(dst_hbm.at[w], idx_v)

        def body(j, carry):
            pltpu.sync_copy(ones_v, cnt_sh.at[idx_v.at[j]], add=True)
            return carry

        lax.fori_loop(0, _NCHUNK, body, 0)
        plsc.subcore_barrier()
        pltpu.sync_copy(cnt_sh.at[pl.ds(sid * _STRIPE, _STRIPE)],
                        out_hbm.at[pl.ds(cid * _ND + sid * _STRIPE, _STRIPE)])

    return sc_gather, sc_scatter, sc_count


def _sc_gather(h, src3, dst3):
    return _sc_kernels()[0](h, src3, dst3)


def _sc_scatter(msg, dst3, zeros_nd):
    return _sc_kernels()[1](msg, dst3, zeros_nd)


def _sc_count(dst3, zeros_nd, ones_ch):
    return _sc_kernels()[2](dst3, zeros_nd, ones_ch)


def kernel(x, edge_index, edge_attr, W1, b1, Wout, bout, root_param, kbias,
           ps0_W, ps0_b, ps0_r, ps1_W, ps1_b, ps1_r, ps2_W, ps2_b, ps2_r,
           psout_W, psout_b, psout_r, bn_g, bn_b, dW1, db1, dW2, db2, dW3,
           db3):
    pad = _E_PAD - _E
    src3 = jnp.concatenate(
        [edge_index[0], jnp.zeros((pad,), jnp.int32)]).reshape(_NW, _NCHUNK, _CH)
    dst3 = jnp.concatenate(
        [edge_index[1], jnp.full((pad,), _DUMMY, jnp.int32)]
    ).reshape(_NW, _NCHUNK, _CH)
    attr_pad = jnp.concatenate([edge_attr, jnp.zeros((pad, 1), _f32)])
    zeros_nd = jnp.zeros((_ND, _W), _f32)
    ones_ch = jnp.ones((_CH, _W), _f32)

    eye = jnp.eye(_W, dtype=_f32)
    rep = jnp.kron(eye, jnp.ones((1, _W), _f32))
    red = jnp.kron(jnp.ones((_W, 1), _f32), eye)

    b1r = b1.reshape(1, _W)
    kbr = kbias.reshape(1, _W)
    boutr = bout.reshape(1, 128)
    bngr = bn_g.reshape(1, 64)
    bnbr = bn_b.reshape(1, 64)
    w0 = ps0_W.reshape(4, 64)
    d1r = dW1.reshape(1, 128)
    e1r = db1.reshape(1, 128)
    e2r = db2.reshape(1, 128)
    e3r = db3.reshape(1, 256)

    h0 = _tc_h0(x, W1, b1r)
    cnts = _sc_count(dst3, zeros_nd, ones_ch)
    st1 = _tc_stats1(edge_attr, w0, ps0_b, ps0_r, ps1_W, ps1_b, ps1_r)
    st2 = _tc_stats2(edge_attr, w0, ps0_b, ps0_r, ps1_W, ps1_b, ps1_r,
                     st1, bngr, bnbr, ps2_W, ps2_b, ps2_r)
    wk, wsum = _tc_weights(attr_pad, w0, ps0_b, ps0_r, ps1_W, ps1_b, ps1_r,
                           st1, bngr, bnbr, ps2_W, ps2_b, ps2_r, st2,
                           psout_W, psout_b, psout_r,
                           d1r, e1r, dW2, e2r, dW3, e3r)
    c0 = cnts[:_N]
    c1 = cnts[_ND:_ND + _N]

    h = h0
    for layer in range(2):
        xj, xi = _sc_gather(h, src3, dst3)
        msg = _tc_msg(xj, xi, wk, wsum, rep, red)
        parts = _sc_scatter(msg, dst3, zeros_nd)
        p0 = parts[:_N]
        p1 = parts[_ND:_ND + _N]
        if layer == 0:
            h = _tc_update(p0, p1, c0, c1, h, root_param, kbr)
        else:
            h = _tc_update_out(p0, p1, c0, c1, h, root_param, kbr, Wout,
                               boutr)
    return h


# speedup vs baseline: 2.7539x; 1.0923x over previous
import functools

import jax
import jax.numpy as jnp
from jax import lax
from jax.experimental import pallas as pl
from jax.experimental.pallas import tpu as pltpu
from jax.experimental.pallas import tpu_sc as plsc

_N = 10000
_E = 160000
_W = 16
_NC = 2
_NS = 16
_NW = _NC * _NS
_CH = 128
_NCHUNK = 40
_EW = _CH * _NCHUNK
_E_PAD = _EW * _NW
_ND = 10240
_STRIPE = _ND // _NS
_DUMMY = _N

_f32 = jnp.float32


def _leaky(x):
    return jnp.where(x >= 0, x, 0.1 * x)


def _ps_combine(y, r_ref, k):
    y0 = y[:, 0:k]
    y1 = _leaky(y[:, k:2 * k])
    y2 = _leaky(y[:, 2 * k:3 * k])
    y3 = _leaky(y[:, 3 * k:4 * k])
    return (r_ref[:, 0:k] * y0 + r_ref[:, k:2 * k] * y1
            + r_ref[:, 2 * k:3 * k] * (y2 * y2)
            + r_ref[:, 3 * k:4 * k] * (y3 * y3 * y3))


def _ps_scalar(a, w_ref, b_ref, r_ref):
    y = a * w_ref[...] + b_ref[...]
    return _ps_combine(y, r_ref, 64)


def _ps_mat(x, w_ref, b_ref, r_ref, k):
    y = jnp.dot(x.astype(jnp.bfloat16), w_ref[...],
                preferred_element_type=_f32) + b_ref[...]
    return _ps_combine(y, r_ref, k)


def _bn_apply(x, st_ref, g_ref, b_ref):
    return ((x - st_ref[0:1, :]) * lax.rsqrt(st_ref[1:2, :] + 1e-5)
            * g_ref[0:1, :] + b_ref[0:1, :])


def _tc_h0(x, w1, b1):
    b = 2000

    def body(x_ref, w_ref, b_ref, o_ref):
        o_ref[...] = (jnp.dot(x_ref[...], w_ref[...], preferred_element_type=_f32)
                      + b_ref[...])

    return pl.pallas_call(
        body,
        grid=(_N // b,),
        in_specs=[pl.BlockSpec((b, 128), lambda i: (i, 0)),
                  pl.BlockSpec((128, _W), lambda i: (0, 0)),
                  pl.BlockSpec((1, _W), lambda i: (0, 0))],
        out_specs=pl.BlockSpec((b, _W), lambda i: (i, 0)),
        out_shape=jax.ShapeDtypeStruct((_N, _W), _f32),
    )(x, w1, b1)


def _stats_finalize(o_ref, nblk, i):
    @pl.when(i == nblk - 1)
    def _():
        s = o_ref[0:1, :]
        q = o_ref[1:2, :]
        m = s * (1.0 / _E)
        v = q * (1.0 / _E) - m * m
        o_ref[0:1, :] = m
        o_ref[1:2, :] = v


def _tc_stats1(attr, w0, b0, r0, w1, b1, r1):
    b = 1600
    nblk = _E // b

    def body(a_ref, w0r, b0r, r0r, w1r, b1r, r1r, o_ref):
        i = pl.program_id(0)

        @pl.when(i == 0)
        def _():
            o_ref[...] = jnp.zeros((2, 64), _f32)

        kx0 = _ps_scalar(a_ref[...], w0r, b0r, r0r)
        kx1 = _ps_mat(kx0, w1r, b1r, r1r, 64)
        o_ref[0:1, :] += jnp.sum(kx1, axis=0, keepdims=True)
        o_ref[1:2, :] += jnp.sum(kx1 * kx1, axis=0, keepdims=True)
        _stats_finalize(o_ref, nblk, i)

    full = lambda s: pl.BlockSpec(s, lambda i: tuple(0 for _ in s))
    return pl.pallas_call(
        body,
        grid=(nblk,),
        in_specs=[pl.BlockSpec((b, 1), lambda i: (i, 0)),
                  full((1, 256)), full((1, 256)), full((1, 256)),
                  full((64, 256)), full((1, 256)), full((1, 256))],
        out_specs=pl.BlockSpec((2, 64), lambda i: (0, 0)),
        out_shape=jax.ShapeDtypeStruct((2, 64), _f32),
    )(attr, w0, b0, r0, w1, b1, r1)


def _tc_stats2(attr, w0, b0, r0, w1, b1, r1, st1, bng, bnb, w2, b2, r2):
    b = 1600
    nblk = _E // b

    def body(a_ref, w0r, b0r, r0r, w1r, b1r, r1r, st1r, gr, br, w2r, b2r, r2r,
             o_ref):
        i = pl.program_id(0)

        @pl.when(i == 0)
        def _():
            o_ref[...] = jnp.zeros((2, 64), _f32)

        kx0 = _ps_scalar(a_ref[...], w0r, b0r, r0r)
        kx1 = _ps_mat(kx0, w1r, b1r, r1r, 64)
        bn1 = _bn_apply(kx1, st1r, gr, br)
        kx2 = _ps_mat(bn1, w2r, b2r, r2r, 64)
        o_ref[0:1, :] += jnp.sum(kx2, axis=0, keepdims=True)
        o_ref[1:2, :] += jnp.sum(kx2 * kx2, axis=0, keepdims=True)
        _stats_finalize(o_ref, nblk, i)

    full = lambda s: pl.BlockSpec(s, lambda i: tuple(0 for _ in s))
    return pl.pallas_call(
        body,
        grid=(nblk,),
        in_specs=[pl.BlockSpec((b, 1), lambda i: (i, 0)),
                  full((1, 256)), full((1, 256)), full((1, 256)),
                  full((64, 256)), full((1, 256)), full((1, 256)),
                  full((2, 64)), full((1, 64)), full((1, 64)),
                  full((64, 256)), full((1, 256)), full((1, 256))],
        out_specs=pl.BlockSpec((2, 64), lambda i: (0, 0)),
        out_shape=jax.ShapeDtypeStruct((2, 64), _f32),
    )(attr, w0, b0, r0, w1, b1, r1, st1, bng, bnb, w2, b2, r2)


def _tc_weights(attr_pad, w0, b0, r0, w1, b1, r1, st1, bng, bnb,
                w2, b2, r2, st2, wo, bo, ro, dw1, db1, dw2, db2, dw3, db3):
    b = 1024
    nblk = _E_PAD // b

    def body(a_ref, w0r, b0r, r0r, w1r, b1r, r1r, st1r, gr, br,
             w2r, b2r, r2r, st2r, wor, bor, ror, d1r, e1r, d2r, e2r, d3r, e3r,
             wk_ref, ws_ref):
        a = a_ref[...]
        kx0 = _ps_scalar(a, w0r, b0r, r0r)
        kx1 = _ps_mat(kx0, w1r, b1r, r1r, 64)
        bn1 = _bn_apply(kx1, st1r, gr, br)
        kx2 = _ps_mat(bn1, w2r, b2r, r2r, 64)
        bn2 = _bn_apply(kx2, st2r, gr, br)
        wk = _ps_mat(bn2, wor, bor, ror, 256)
        hd = jnp.maximum(a * d1r[...] + e1r[...], 0.0)
        hd = jnp.maximum(
            jnp.dot(hd.astype(jnp.bfloat16), d2r[...],
                    preferred_element_type=_f32) + e2r[...], 0.0)
        wop = jnp.dot(hd.astype(jnp.bfloat16), d3r[...],
                      preferred_element_type=_f32) + e3r[...]
        wk_ref[...] = wk.astype(jnp.bfloat16)
        ws_ref[...] = (wk + wop).astype(jnp.bfloat16)

    full = lambda s: pl.BlockSpec(s, lambda i: tuple(0 for _ in s))
    return pl.pallas_call(
        body,
        grid=(nblk,),
        in_specs=[pl.BlockSpec((b, 1), lambda i: (i, 0)),
                  full((1, 256)), full((1, 256)), full((1, 256)),
                  full((64, 256)), full((1, 256)), full((1, 256)),
                  full((2, 64)), full((1, 64)), full((1, 64)),
                  full((64, 256)), full((1, 256)), full((1, 256)),
                  full((2, 64)),
                  full((64, 1024)), full((1, 1024)), full((1, 1024)),
                  full((1, 128)), full((1, 128)),
                  full((128, 128)), full((1, 128)),
                  full((128, 256)), full((1, 256))],
        out_specs=[pl.BlockSpec((b, 256), lambda i: (i, 0)),
                   pl.BlockSpec((b, 256), lambda i: (i, 0))],
        out_shape=[jax.ShapeDtypeStruct((_E_PAD, 256), jnp.bfloat16),
                   jax.ShapeDtypeStruct((_E_PAD, 256), jnp.bfloat16)],
    )(attr_pad, w0, b0, r0, w1, b1, r1, st1, bng, bnb,
      w2, b2, r2, st2, wo, bo, ro, dw1, db1, dw2, db2, dw3, db3)


def _tc_msg(xj, xi, wk, wsum, rep, red):
    b = 2048
    nblk = _E_PAD // b

    def body(xj_ref, xi_ref, wk_ref, ws_ref, rep_ref, red_ref, o_ref):
        xjr = jnp.dot(xj_ref[...], rep_ref[...], preferred_element_type=_f32)
        xir = jnp.dot(xi_ref[...], rep_ref[...], preferred_element_type=_f32)
        t = (xjr * ws_ref[...].astype(_f32)
             - xir * wk_ref[...].astype(_f32))
        o_ref[...] = jnp.dot(t, red_ref[...], preferred_element_type=_f32)

    full = lambda s: pl.BlockSpec(s, lambda i: tuple(0 for _ in s))
    return pl.pallas_call(
        body,
        grid=(nblk,),
        in_specs=[pl.BlockSpec((b, _W), lambda i: (i, 0)),
                  pl.BlockSpec((b, _W), lambda i: (i, 0)),
                  pl.BlockSpec((b, 256), lambda i: (i, 0)),
                  pl.BlockSpec((b, 256), lambda i: (i, 0)),
                  full((_W, 256)), full((256, _W))],
        out_specs=pl.BlockSpec((b, _W), lambda i: (i, 0)),
        out_shape=jax.ShapeDtypeStruct((_E_PAD, _W), _f32),
    )(xj, xi, wk, wsum, rep, red)


def _tc_update(p0, p1, c0, c1, h, root, kb):
    b = 2000

    def body(p0r, p1r, c0r, c1r, hr, rtr, kbr, o_ref):
        agg = (p0r[...] + p1r[...]) / jnp.maximum(c0r[...] + c1r[...], 1.0)
        hv = jnp.dot(hr[...], rtr[...], preferred_element_type=_f32)
        o_ref[...] = jnp.maximum(agg + hv + kbr[...], 0.0)

    full = lambda s: pl.BlockSpec(s, lambda i: tuple(0 for _ in s))
    blk = pl.BlockSpec((b, _W), lambda i: (i, 0))
    return pl.pallas_call(
        body,
        grid=(_N // b,),
        in_specs=[blk, blk, blk, blk, blk, full((_W, _W)), full((1, _W))],
        out_specs=pl.BlockSpec((b, _W), lambda i: (i, 0)),
        out_shape=jax.ShapeDtypeStruct((_N, _W), _f32),
    )(p0, p1, c0, c1, h, root, kb)


def _tc_update_out(p0, p1, c0, c1, h, root, kb, wout, bout):
    b = 2000

    def body(p0r, p1r, c0r, c1r, hr, rtr, kbr, wor, bor, o_ref):
        agg = (p0r[...] + p1r[...]) / jnp.maximum(c0r[...] + c1r[...], 1.0)
        hv = jnp.dot(hr[...], rtr[...], preferred_element_type=_f32)
        hn = jnp.maximum(agg + hv + kbr[...], 0.0)
        o_ref[...] = (jnp.dot(hn, wor[...], preferred_element_type=_f32)
                      + bor[...])

    full = lambda s: pl.BlockSpec(s, lambda i: tuple(0 for _ in s))
    blk = pl.BlockSpec((b, _W), lambda i: (i, 0))
    return pl.pallas_call(
        body,
        grid=(_N // b,),
        in_specs=[blk, blk, blk, blk, blk, full((_W, _W)), full((1, _W)),
                  full((_W, 128)), full((1, 128))],
        out_specs=pl.BlockSpec((b, 128), lambda i: (i, 0)),
        out_shape=jax.ShapeDtypeStruct((_N, 128), _f32),
    )(p0, p1, c0, c1, h, root, kb, wout, bout)


@functools.cache
def _sc_kernels():
    mesh = plsc.VectorSubcoreMesh(core_axis_name="c", subcore_axis_name="s")
    params = pltpu.CompilerParams(use_tc_tiling_on_sc=False)

    @functools.partial(
        pl.kernel,
        out_type=[jax.ShapeDtypeStruct((_E_PAD, _W), _f32),
                  jax.ShapeDtypeStruct((_E_PAD, _W), _f32)],
        mesh=mesh,
        compiler_params=params,
        scratch_types=[pltpu.VMEM((_NCHUNK, _CH), jnp.int32),
                       pltpu.VMEM((_EW, _W), _f32),
                       pltpu.SemaphoreType.DMA],
    )
    def sc_gather(h_hbm, src_hbm, dst_hbm, xj_hbm, xi_hbm, idx_v, rows_v,
                  sem):
        w = lax.axis_index("s") * _NC + lax.axis_index("c")
        for idx_hbm, out_hbm in ((src_hbm, xj_hbm), (dst_hbm, xi_hbm)):
            pltpu.sync_copy(idx_hbm.at[w], idx_v)

            def body(j, carry):
                pltpu.async_copy(h_hbm.at[idx_v.at[j]],
                                 rows_v.at[pl.ds(j * _CH, _CH)], sem).wait()
                return carry

            lax.fori_loop(0, _NCHUNK, body, 0)
            pltpu.sync_copy(rows_v, out_hbm.at[pl.ds(w * _EW, _EW)])

    @functools.partial(
        pl.kernel,
        out_type=jax.ShapeDtypeStruct((_NC * _ND, _W), _f32),
        mesh=mesh,
        compiler_params=params,
        scratch_types=[pltpu.VMEM((_NCHUNK, _CH), jnp.int32),
                       pltpu.VMEM((_EW, _W), _f32),
                       pltpu.VMEM_SHARED((_ND, _W), _f32)],
    )
    def sc_scatter(msg_hbm, dst_hbm, zeros_hbm, out_hbm, idx_v, msg_v,
                   agg_sh):
        cid = lax.axis_index("c")
        sid = lax.axis_index("s")
        w = sid * _NC + cid
        pltpu.sync_copy(zeros_hbm.at[pl.ds(sid * _STRIPE, _STRIPE)],
                        agg_sh.at[pl.ds(sid * _STRIPE, _STRIPE)])
        plsc.subcore_barrier()
        pltpu.sync_copy(dst_hbm.at[w], idx_v)
        pltpu.sync_copy(msg_hbm.at[pl.ds(w * _EW, _EW)], msg_v)

        def body(j, carry):
            pltpu.sync_copy(msg_v.at[pl.ds(j * _CH, _CH)],
                            agg_sh.at[idx_v.at[j]], add=True)
            return carry

        lax.fori_loop(0, _NCHUNK, body, 0)
        plsc.subcore_barrier()
        pltpu.sync_copy(agg_sh.at[pl.ds(sid * _STRIPE, _STRIPE)],
                        out_hbm.at[pl.ds(cid * _ND + sid * _STRIPE, _STRIPE)])

    @functools.partial(
        pl.kernel,
        out_type=jax.ShapeDtypeStruct((_NC * _ND, _W), _f32),
        mesh=mesh,
        compiler_params=params,
        scratch_types=[pltpu.VMEM((_NCHUNK, _CH), jnp.int32),
                       pltpu.VMEM((_CH, _W), _f32),
                       pltpu.VMEM_SHARED((_ND, _W), _f32)],
    )
    def sc_count(dst_hbm, zeros_hbm, ones_hbm, out_hbm, idx_v, ones_v,
                 cnt_sh):
        cid = lax.axis_index("c")
        sid = lax.axis_index("s")
        w = sid * _NC + cid
        pltpu.sync_copy(zeros_hbm.at[pl.ds(sid * _STRIPE, _STRIPE)],
                        cnt_sh.at[pl.ds(sid * _STRIPE, _STRIPE)])
        pltpu.sync_copy(ones_hbm, ones_v)
        plsc.subcore_barrier()
        pltpu.sync_copy(dst_hbm.at[w], idx_v)

        def body(j, carry):
            pltpu.sync_copy(ones_v, cnt_sh.at[idx_v.at[j]], add=True)
            return carry

        lax.fori_loop(0, _NCHUNK, body, 0)
        plsc.subcore_barrier()
        pltpu.sync_copy(cnt_sh.at[pl.ds(sid * _STRIPE, _STRIPE)],
                        out_hbm.at[pl.ds(cid * _ND + sid * _STRIPE, _STRIPE)])

    return sc_gather, sc_scatter, sc_count


def _sc_gather(h, src3, dst3):
    return _sc_kernels()[0](h, src3, dst3)


def _sc_scatter(msg, dst3, zeros_nd):
    return _sc_kernels()[1](msg, dst3, zeros_nd)


def _sc_count(dst3, zeros_nd, ones_ch):
    return _sc_kernels()[2](dst3, zeros_nd, ones_ch)


def kernel(x, edge_index, edge_attr, W1, b1, Wout, bout, root_param, kbias,
           ps0_W, ps0_b, ps0_r, ps1_W, ps1_b, ps1_r, ps2_W, ps2_b, ps2_r,
           psout_W, psout_b, psout_r, bn_g, bn_b, dW1, db1, dW2, db2, dW3,
           db3):
    pad = _E_PAD - _E
    src3 = jnp.concatenate(
        [edge_index[0], jnp.zeros((pad,), jnp.int32)]).reshape(_NW, _NCHUNK, _CH)
    dst3 = jnp.concatenate(
        [edge_index[1], jnp.full((pad,), _DUMMY, jnp.int32)]
    ).reshape(_NW, _NCHUNK, _CH)
    attr_pad = jnp.concatenate([edge_attr, jnp.zeros((pad, 1), _f32)])
    zeros_nd = jnp.zeros((_ND, _W), _f32)
    ones_ch = jnp.ones((_CH, _W), _f32)

    eye = jnp.eye(_W, dtype=_f32)
    rep = jnp.kron(eye, jnp.ones((1, _W), _f32))
    red = jnp.kron(jnp.ones((_W, 1), _f32), eye)

    b1r = b1.reshape(1, _W)
    kbr = kbias.reshape(1, _W)
    boutr = bout.reshape(1, 128)
    bngr = bn_g.reshape(1, 64)
    bnbr = bn_b.reshape(1, 64)
    bf16 = jnp.bfloat16
    w0 = ps0_W.reshape(1, 256)
    b0 = ps0_b.reshape(1, 256)
    r0 = ps0_r.reshape(1, 256)
    w1c = jnp.transpose(ps1_W, (1, 0, 2)).reshape(64, 256).astype(bf16)
    b1c = ps1_b.reshape(1, 256)
    r1c = ps1_r.reshape(1, 256)
    w2c = jnp.transpose(ps2_W, (1, 0, 2)).reshape(64, 256).astype(bf16)
    b2c = ps2_b.reshape(1, 256)
    r2c = ps2_r.reshape(1, 256)
    woc = jnp.transpose(psout_W, (1, 0, 2)).reshape(64, 1024).astype(bf16)
    boc = psout_b.reshape(1, 1024)
    roc = psout_r.reshape(1, 1024)
    d1r = dW1.reshape(1, 128)
    e1r = db1.reshape(1, 128)
    e2r = db2.reshape(1, 128)
    e3r = db3.reshape(1, 256)
    dw2b = dW2.astype(bf16)
    dw3b = dW3.astype(bf16)

    h0 = _tc_h0(x, W1, b1r)
    cnts = _sc_count(dst3, zeros_nd, ones_ch)
    st1 = _tc_stats1(edge_attr, w0, b0, r0, w1c, b1c, r1c)
    st2 = _tc_stats2(edge_attr, w0, b0, r0, w1c, b1c, r1c,
                     st1, bngr, bnbr, w2c, b2c, r2c)
    wk, wsum = _tc_weights(attr_pad, w0, b0, r0, w1c, b1c, r1c,
                           st1, bngr, bnbr, w2c, b2c, r2c, st2,
                           woc, boc, roc,
                           d1r, e1r, dw2b, e2r, dw3b, e3r)
    c0 = cnts[:_N]
    c1 = cnts[_ND:_ND + _N]

    h = h0
    for layer in range(2):
        xj, xi = _sc_gather(h, src3, dst3)
        msg = _tc_msg(xj, xi, wk, wsum, rep, red)
        parts = _sc_scatter(msg, dst3, zeros_nd)
        p0 = parts[:_N]
        p1 = parts[_ND:_ND + _N]
        if layer == 0:
            h = _tc_update(p0, p1, c0, c1, h, root_param, kbr)
        else:
            h = _tc_update_out(p0, p1, c0, c1, h, root_param, kbr, Wout,
                               boutr)
    return h


# speedup vs baseline: 2.7611x; 1.0026x over previous
import functools

import jax
import jax.numpy as jnp
from jax import lax
from jax.experimental import pallas as pl
from jax.experimental.pallas import tpu as pltpu
from jax.experimental.pallas import tpu_sc as plsc

_N = 10000
_E = 160000
_W = 16
_NC = 2
_NS = 16
_NW = _NC * _NS
_CH = 128
_NCHUNK = 40
_EW = _CH * _NCHUNK
_E_PAD = _EW * _NW
_ND = 10240
_STRIPE = _ND // _NS
_DUMMY = _N

_f32 = jnp.float32


def _leaky(x):
    return jnp.where(x >= 0, x, 0.1 * x)


def _ps_combine(y, r_ref, k):
    y0 = y[:, 0:k]
    y1 = _leaky(y[:, k:2 * k])
    y2 = _leaky(y[:, 2 * k:3 * k])
    y3 = _leaky(y[:, 3 * k:4 * k])
    return (r_ref[:, 0:k] * y0 + r_ref[:, k:2 * k] * y1
            + r_ref[:, 2 * k:3 * k] * (y2 * y2)
            + r_ref[:, 3 * k:4 * k] * (y3 * y3 * y3))


def _ps_scalar(a, w_ref, b_ref, r_ref):
    y = a * w_ref[...] + b_ref[...]
    return _ps_combine(y, r_ref, 64)


def _ps_mat(x, w_ref, b_ref, r_ref, k):
    y = jnp.dot(x.astype(jnp.bfloat16), w_ref[...],
                preferred_element_type=_f32) + b_ref[...]
    return _ps_combine(y, r_ref, k)


def _bn_apply(x, st_ref, g_ref, b_ref):
    return ((x - st_ref[0:1, :]) * lax.rsqrt(st_ref[1:2, :] + 1e-5)
            * g_ref[0:1, :] + b_ref[0:1, :])


def _tc_h0(x, w1, b1):
    b = 2000

    def body(x_ref, w_ref, b_ref, o_ref):
        o_ref[...] = (jnp.dot(x_ref[...], w_ref[...], preferred_element_type=_f32)
                      + b_ref[...])

    return pl.pallas_call(
        body,
        grid=(_N // b,),
        in_specs=[pl.BlockSpec((b, 128), lambda i: (i, 0)),
                  pl.BlockSpec((128, _W), lambda i: (0, 0)),
                  pl.BlockSpec((1, _W), lambda i: (0, 0))],
        out_specs=pl.BlockSpec((b, _W), lambda i: (i, 0)),
        out_shape=jax.ShapeDtypeStruct((_N, _W), _f32),
    )(x, w1, b1)


def _stats_finalize(o_ref, nblk, i):
    @pl.when(i == nblk - 1)
    def _():
        s = o_ref[0:1, :]
        q = o_ref[1:2, :]
        m = s * (1.0 / _E)
        v = q * (1.0 / _E) - m * m
        o_ref[0:1, :] = m
        o_ref[1:2, :] = v


def _tc_stats1(attr, w0, b0, r0, w1, b1, r1):
    b = 1600
    nblk = _E // b

    def body(a_ref, w0r, b0r, r0r, w1r, b1r, r1r, o_ref):
        i = pl.program_id(0)

        @pl.when(i == 0)
        def _():
            o_ref[...] = jnp.zeros((2, 64), _f32)

        kx0 = _ps_scalar(a_ref[...], w0r, b0r, r0r)
        kx1 = _ps_mat(kx0, w1r, b1r, r1r, 64)
        o_ref[0:1, :] += jnp.sum(kx1, axis=0, keepdims=True)
        o_ref[1:2, :] += jnp.sum(kx1 * kx1, axis=0, keepdims=True)
        _stats_finalize(o_ref, nblk, i)

    full = lambda s: pl.BlockSpec(s, lambda i: tuple(0 for _ in s))
    return pl.pallas_call(
        body,
        grid=(nblk,),
        in_specs=[pl.BlockSpec((b, 1), lambda i: (i, 0)),
                  full((1, 256)), full((1, 256)), full((1, 256)),
                  full((64, 256)), full((1, 256)), full((1, 256))],
        out_specs=pl.BlockSpec((2, 64), lambda i: (0, 0)),
        out_shape=jax.ShapeDtypeStruct((2, 64), _f32),
    )(attr, w0, b0, r0, w1, b1, r1)


def _tc_stats2(attr, w0, b0, r0, w1, b1, r1, st1, bng, bnb, w2, b2, r2):
    b = 1600
    nblk = _E // b

    def body(a_ref, w0r, b0r, r0r, w1r, b1r, r1r, st1r, gr, br, w2r, b2r, r2r,
             o_ref):
        i = pl.program_id(0)

        @pl.when(i == 0)
        def _():
            o_ref[...] = jnp.zeros((2, 64), _f32)

        kx0 = _ps_scalar(a_ref[...], w0r, b0r, r0r)
        kx1 = _ps_mat(kx0, w1r, b1r, r1r, 64)
        bn1 = _bn_apply(kx1, st1r, gr, br)
        kx2 = _ps_mat(bn1, w2r, b2r, r2r, 64)
        o_ref[0:1, :] += jnp.sum(kx2, axis=0, keepdims=True)
        o_ref[1:2, :] += jnp.sum(kx2 * kx2, axis=0, keepdims=True)
        _stats_finalize(o_ref, nblk, i)

    full = lambda s: pl.BlockSpec(s, lambda i: tuple(0 for _ in s))
    return pl.pallas_call(
        body,
        grid=(nblk,),
        in_specs=[pl.BlockSpec((b, 1), lambda i: (i, 0)),
                  full((1, 256)), full((1, 256)), full((1, 256)),
                  full((64, 256)), full((1, 256)), full((1, 256)),
                  full((2, 64)), full((1, 64)), full((1, 64)),
                  full((64, 256)), full((1, 256)), full((1, 256))],
        out_specs=pl.BlockSpec((2, 64), lambda i: (0, 0)),
        out_shape=jax.ShapeDtypeStruct((2, 64), _f32),
    )(attr, w0, b0, r0, w1, b1, r1, st1, bng, bnb, w2, b2, r2)


def _tc_weights(attr_pad, w0, b0, r0, w1, b1, r1, st1, bng, bnb,
                w2, b2, r2, st2, wo, bo, ro, dw1, db1, dw2, db2, dw3, db3):
    b = 1024
    nblk = _E_PAD // b

    def body(a_ref, w0r, b0r, r0r, w1r, b1r, r1r, st1r, gr, br,
             w2r, b2r, r2r, st2r, wor, bor, ror, d1r, e1r, d2r, e2r, d3r, e3r,
             wk_ref, ws_ref):
        a = a_ref[...]
        kx0 = _ps_scalar(a, w0r, b0r, r0r)
        kx1 = _ps_mat(kx0, w1r, b1r, r1r, 64)
        bn1 = _bn_apply(kx1, st1r, gr, br)
        kx2 = _ps_mat(bn1, w2r, b2r, r2r, 64)
        bn2 = _bn_apply(kx2, st2r, gr, br)
        wk = _ps_mat(bn2, wor, bor, ror, 256)
        hd = jnp.maximum(a * d1r[...] + e1r[...], 0.0)
        hd = jnp.maximum(
            jnp.dot(hd.astype(jnp.bfloat16), d2r[...],
                    preferred_element_type=_f32) + e2r[...], 0.0)
        wop = jnp.dot(hd.astype(jnp.bfloat16), d3r[...],
                      preferred_element_type=_f32) + e3r[...]
        wk_ref[...] = wk.astype(jnp.bfloat16)
        ws_ref[...] = (wk + wop).astype(jnp.bfloat16)

    full = lambda s: pl.BlockSpec(s, lambda i: tuple(0 for _ in s))
    return pl.pallas_call(
        body,
        grid=(nblk,),
        in_specs=[pl.BlockSpec((b, 1), lambda i: (i, 0)),
                  full((1, 256)), full((1, 256)), full((1, 256)),
                  full((64, 256)), full((1, 256)), full((1, 256)),
                  full((2, 64)), full((1, 64)), full((1, 64)),
                  full((64, 256)), full((1, 256)), full((1, 256)),
                  full((2, 64)),
                  full((64, 1024)), full((1, 1024)), full((1, 1024)),
                  full((1, 128)), full((1, 128)),
                  full((128, 128)), full((1, 128)),
                  full((128, 256)), full((1, 256))],
        out_specs=[pl.BlockSpec((b, 256), lambda i: (i, 0)),
                   pl.BlockSpec((b, 256), lambda i: (i, 0))],
        out_shape=[jax.ShapeDtypeStruct((_E_PAD, 256), jnp.bfloat16),
                   jax.ShapeDtypeStruct((_E_PAD, 256), jnp.bfloat16)],
    )(attr_pad, w0, b0, r0, w1, b1, r1, st1, bng, bnb,
      w2, b2, r2, st2, wo, bo, ro, dw1, db1, dw2, db2, dw3, db3)


def _tc_msg(xj, xi, wk, wsum, rep, red):
    b = 2048
    nblk = _E_PAD // b

    def body(xj_ref, xi_ref, wk_ref, ws_ref, rep_ref, red_ref, o_ref):
        xjr = jnp.dot(xj_ref[...], rep_ref[...], preferred_element_type=_f32)
        xir = jnp.dot(xi_ref[...], rep_ref[...], preferred_element_type=_f32)
        t = (xjr * ws_ref[...].astype(_f32)
             - xir * wk_ref[...].astype(_f32))
        o_ref[...] = jnp.dot(t, red_ref[...], preferred_element_type=_f32)

    full = lambda s: pl.BlockSpec(s, lambda i: tuple(0 for _ in s))
    return pl.pallas_call(
        body,
        grid=(nblk,),
        in_specs=[pl.BlockSpec((b, _W), lambda i: (i, 0)),
                  pl.BlockSpec((b, _W), lambda i: (i, 0)),
                  pl.BlockSpec((b, 256), lambda i: (i, 0)),
                  pl.BlockSpec((b, 256), lambda i: (i, 0)),
                  full((_W, 256)), full((256, _W))],
        out_specs=pl.BlockSpec((b, _W), lambda i: (i, 0)),
        out_shape=jax.ShapeDtypeStruct((_E_PAD, _W), _f32),
    )(xj, xi, wk, wsum, rep, red)


def _tc_update(p0, p1, c0, c1, h, root, kb):
    b = 2000

    def body(p0r, p1r, c0r, c1r, hr, rtr, kbr, o_ref):
        agg = (p0r[...] + p1r[...]) / jnp.maximum(c0r[...] + c1r[...], 1.0)
        hv = jnp.dot(hr[...], rtr[...], preferred_element_type=_f32)
        o_ref[...] = jnp.maximum(agg + hv + kbr[...], 0.0)

    full = lambda s: pl.BlockSpec(s, lambda i: tuple(0 for _ in s))
    blk = pl.BlockSpec((b, _W), lambda i: (i, 0))
    return pl.pallas_call(
        body,
        grid=(_N // b,),
        in_specs=[blk, blk, blk, blk, blk, full((_W, _W)), full((1, _W))],
        out_specs=pl.BlockSpec((b, _W), lambda i: (i, 0)),
        out_shape=jax.ShapeDtypeStruct((_N, _W), _f32),
    )(p0, p1, c0, c1, h, root, kb)


def _tc_update_out(p0, p1, c0, c1, h, root, kb, wout, bout):
    b = 2000

    def body(p0r, p1r, c0r, c1r, hr, rtr, kbr, wor, bor, o_ref):
        agg = (p0r[...] + p1r[...]) / jnp.maximum(c0r[...] + c1r[...], 1.0)
        hv = jnp.dot(hr[...], rtr[...], preferred_element_type=_f32)
        hn = jnp.maximum(agg + hv + kbr[...], 0.0)
        o_ref[...] = (jnp.dot(hn, wor[...], preferred_element_type=_f32)
                      + bor[...])

    full = lambda s: pl.BlockSpec(s, lambda i: tuple(0 for _ in s))
    blk = pl.BlockSpec((b, _W), lambda i: (i, 0))
    return pl.pallas_call(
        body,
        grid=(_N // b,),
        in_specs=[blk, blk, blk, blk, blk, full((_W, _W)), full((1, _W)),
                  full((_W, 128)), full((1, 128))],
        out_specs=pl.BlockSpec((b, 128), lambda i: (i, 0)),
        out_shape=jax.ShapeDtypeStruct((_N, 128), _f32),
    )(p0, p1, c0, c1, h, root, kb, wout, bout)


@functools.cache
def _sc_kernels():
    mesh = plsc.VectorSubcoreMesh(core_axis_name="c", subcore_axis_name="s")
    params = pltpu.CompilerParams(use_tc_tiling_on_sc=False)

    @functools.partial(
        pl.kernel,
        out_type=[jax.ShapeDtypeStruct((_E_PAD, _W), _f32),
                  jax.ShapeDtypeStruct((_E_PAD, _W), _f32)],
        mesh=mesh,
        compiler_params=params,
        scratch_types=[pltpu.VMEM((_NCHUNK, _CH), jnp.int32),
                       pltpu.VMEM((_EW, _W), _f32),
                       pltpu.SemaphoreType.DMA],
    )
    def sc_gather(h_hbm, src_hbm, dst_hbm, xj_hbm, xi_hbm, idx_v, rows_v,
                  sem):
        w = lax.axis_index("s") * _NC + lax.axis_index("c")
        for idx_hbm, out_hbm in ((src_hbm, xj_hbm), (dst_hbm, xi_hbm)):
            pltpu.sync_copy(idx_hbm.at[w], idx_v)

            def fire(j, carry):
                pltpu.async_copy(h_hbm.at[idx_v.at[j]],
                                 rows_v.at[pl.ds(j * _CH, _CH)], sem)
                return carry

            def drain(j, carry):
                pltpu.make_async_copy(
                    h_hbm.at[idx_v.at[j]],
                    rows_v.at[pl.ds(j * _CH, _CH)], sem).wait()
                return carry

            lax.fori_loop(0, _NCHUNK, fire, 0)
            lax.fori_loop(0, _NCHUNK, drain, 0)
            pltpu.sync_copy(rows_v, out_hbm.at[pl.ds(w * _EW, _EW)])

    @functools.partial(
        pl.kernel,
        out_type=jax.ShapeDtypeStruct((_NC * _ND, _W), _f32),
        mesh=mesh,
        compiler_params=params,
        scratch_types=[pltpu.VMEM((_NCHUNK, _CH), jnp.int32),
                       pltpu.VMEM((_EW, _W), _f32),
                       pltpu.VMEM_SHARED((_ND, _W), _f32),
                       pltpu.SemaphoreType.DMA],
    )
    def sc_scatter(msg_hbm, dst_hbm, zeros_hbm, out_hbm, idx_v, msg_v,
                   agg_sh, sem):
        cid = lax.axis_index("c")
        sid = lax.axis_index("s")
        w = sid * _NC + cid
        pltpu.sync_copy(zeros_hbm.at[pl.ds(sid * _STRIPE, _STRIPE)],
                        agg_sh.at[pl.ds(sid * _STRIPE, _STRIPE)])
        plsc.subcore_barrier()
        pltpu.sync_copy(dst_hbm.at[w], idx_v)
        pltpu.sync_copy(msg_hbm.at[pl.ds(w * _EW, _EW)], msg_v)

        def fire(j, carry):
            pltpu.async_copy(msg_v.at[pl.ds(j * _CH, _CH)],
                             agg_sh.at[idx_v.at[j]], sem, add=True)
            return carry

        def drain(j, carry):
            pltpu.make_async_copy(msg_v.at[pl.ds(j * _CH, _CH)],
                                  agg_sh.at[idx_v.at[j]], sem).wait()
            return carry

        lax.fori_loop(0, _NCHUNK, fire, 0)
        lax.fori_loop(0, _NCHUNK, drain, 0)
        plsc.subcore_barrier()
        pltpu.sync_copy(agg_sh.at[pl.ds(sid * _STRIPE, _STRIPE)],
                        out_hbm.at[pl.ds(cid * _ND + sid * _STRIPE, _STRIPE)])

    @functools.partial(
        pl.kernel,
        out_type=jax.ShapeDtypeStruct((_NC * _ND, _W), _f32),
        mesh=mesh,
        compiler_params=params,
        scratch_types=[pltpu.VMEM((_NCHUNK, _CH), jnp.int32),
                       pltpu.VMEM((_CH, _W), _f32),
                       pltpu.VMEM_SHARED((_ND, _W), _f32),
                       pltpu.SemaphoreType.DMA],
    )
    def sc_count(dst_hbm, zeros_hbm, ones_hbm, out_hbm, idx_v, ones_v,
                 cnt_sh, sem):
        cid = lax.axis_index("c")
        sid = lax.axis_index("s")
        w = sid * _NC + cid
        pltpu.sync_copy(zeros_hbm.at[pl.ds(sid * _STRIPE, _STRIPE)],
                        cnt_sh.at[pl.ds(sid * _STRIPE, _STRIPE)])
        pltpu.sync_copy(ones_hbm, ones_v)
        plsc.subcore_barrier()
        pltpu.sync_copy(dst_hbm.at[w], idx_v)

        def fire(j, carry):
            pltpu.async_copy(ones_v, cnt_sh.at[idx_v.at[j]], sem, add=True)
            return carry

        def drain(j, carry):
            pltpu.make_async_copy(ones_v, cnt_sh.at[idx_v.at[j]], sem).wait()
            return carry

        lax.fori_loop(0, _NCHUNK, fire, 0)
        lax.fori_loop(0, _NCHUNK, drain, 0)
        plsc.subcore_barrier()
        pltpu.sync_copy(cnt_sh.at[pl.ds(sid * _STRIPE, _STRIPE)],
                        out_hbm.at[pl.ds(cid * _ND + sid * _STRIPE, _STRIPE)])

    return sc_gather, sc_scatter, sc_count


def _sc_gather(h, src3, dst3):
    return _sc_kernels()[0](h, src3, dst3)


def _sc_scatter(msg, dst3, zeros_nd):
    return _sc_kernels()[1](msg, dst3, zeros_nd)


def _sc_count(dst3, zeros_nd, ones_ch):
    return _sc_kernels()[2](dst3, zeros_nd, ones_ch)


def kernel(x, edge_index, edge_attr, W1, b1, Wout, bout, root_param, kbias,
           ps0_W, ps0_b, ps0_r, ps1_W, ps1_b, ps1_r, ps2_W, ps2_b, ps2_r,
           psout_W, psout_b, psout_r, bn_g, bn_b, dW1, db1, dW2, db2, dW3,
           db3):
    pad = _E_PAD - _E
    src3 = jnp.concatenate(
        [edge_index[0], jnp.zeros((pad,), jnp.int32)]).reshape(_NW, _NCHUNK, _CH)
    dst3 = jnp.concatenate(
        [edge_index[1], jnp.full((pad,), _DUMMY, jnp.int32)]
    ).reshape(_NW, _NCHUNK, _CH)
    attr_pad = jnp.concatenate([edge_attr, jnp.zeros((pad, 1), _f32)])
    zeros_nd = jnp.zeros((_ND, _W), _f32)
    ones_ch = jnp.ones((_CH, _W), _f32)

    eye = jnp.eye(_W, dtype=_f32)
    rep = jnp.kron(eye, jnp.ones((1, _W), _f32))
    red = jnp.kron(jnp.ones((_W, 1), _f32), eye)

    b1r = b1.reshape(1, _W)
    kbr = kbias.reshape(1, _W)
    boutr = bout.reshape(1, 128)
    bngr = bn_g.reshape(1, 64)
    bnbr = bn_b.reshape(1, 64)
    bf16 = jnp.bfloat16
    w0 = ps0_W.reshape(1, 256)
    b0 = ps0_b.reshape(1, 256)
    r0 = ps0_r.reshape(1, 256)
    w1c = jnp.transpose(ps1_W, (1, 0, 2)).reshape(64, 256).astype(bf16)
    b1c = ps1_b.reshape(1, 256)
    r1c = ps1_r.reshape(1, 256)
    w2c = jnp.transpose(ps2_W, (1, 0, 2)).reshape(64, 256).astype(bf16)
    b2c = ps2_b.reshape(1, 256)
    r2c = ps2_r.reshape(1, 256)
    woc = jnp.transpose(psout_W, (1, 0, 2)).reshape(64, 1024).astype(bf16)
    boc = psout_b.reshape(1, 1024)
    roc = psout_r.reshape(1, 1024)
    d1r = dW1.reshape(1, 128)
    e1r = db1.reshape(1, 128)
    e2r = db2.reshape(1, 128)
    e3r = db3.reshape(1, 256)
    dw2b = dW2.astype(bf16)
    dw3b = dW3.astype(bf16)

    h0 = _tc_h0(x, W1, b1r)
    cnts = _sc_count(dst3, zeros_nd, ones_ch)
    st1 = _tc_stats1(edge_attr, w0, b0, r0, w1c, b1c, r1c)
    st2 = _tc_stats2(edge_attr, w0, b0, r0, w1c, b1c, r1c,
                     st1, bngr, bnbr, w2c, b2c, r2c)
    wk, wsum = _tc_weights(attr_pad, w0, b0, r0, w1c, b1c, r1c,
                           st1, bngr, bnbr, w2c, b2c, r2c, st2,
                           woc, boc, roc,
                           d1r, e1r, dw2b, e2r, dw3b, e3r)
    c0 = cnts[:_N]
    c1 = cnts[_ND:_ND + _N]

    h = h0
    for layer in range(2):
        xj, xi = _sc_gather(h, src3, dst3)
        msg = _tc_msg(xj, xi, wk, wsum, rep, red)
        parts = _sc_scatter(msg, dst3, zeros_nd)
        p0 = parts[:_N]
        p1 = parts[_ND:_ND + _N]
        if layer == 0:
            h = _tc_update(p0, p1, c0, c1, h, root_param, kbr)
        else:
            h = _tc_update_out(p0, p1, c0, c1, h, root_param, kbr, Wout,
                               boutr)
    return h


# speedup vs baseline: 3.1670x; 1.1470x over previous
import functools

import jax
import jax.numpy as jnp
from jax import lax
from jax.experimental import pallas as pl
from jax.experimental.pallas import tpu as pltpu
from jax.experimental.pallas import tpu_sc as plsc

_N = 10000
_E = 160000
_W = 16
_NC = 2
_NS = 16
_NW = _NC * _NS
_CH = 128
_NCHUNK = 40
_EW = _CH * _NCHUNK
_E_PAD = _EW * _NW
_EP = _E_PAD // 8
_ND = 10240
_STRIPE = _ND // _NS
_DUMMY = _N
_BM = 1024

_f32 = jnp.float32


def _leaky(x):
    return jnp.where(x >= 0, x, 0.1 * x)


def _ps_combine(y, r_ref, k):
    y0 = y[:, 0:k]
    y1 = _leaky(y[:, k:2 * k])
    y2 = _leaky(y[:, 2 * k:3 * k])
    y3 = _leaky(y[:, 3 * k:4 * k])
    return (r_ref[:, 0:k] * y0 + r_ref[:, k:2 * k] * y1
            + r_ref[:, 2 * k:3 * k] * (y2 * y2)
            + r_ref[:, 3 * k:4 * k] * (y3 * y3 * y3))


def _ps_scalar(a, w_ref, b_ref, r_ref):
    y = a * w_ref[...] + b_ref[...]
    return _ps_combine(y, r_ref, 64)


def _ps_mat(x, w_ref, b_ref, r_ref, k):
    y = jnp.dot(x.astype(jnp.bfloat16), w_ref[...],
                preferred_element_type=_f32) + b_ref[...]
    return _ps_combine(y, r_ref, k)


def _bn_apply(x, st_ref, g_ref, b_ref):
    return ((x - st_ref[0:1, :]) * lax.rsqrt(st_ref[1:2, :] + 1e-5)
            * g_ref[0:1, :] + b_ref[0:1, :])


def _tc_h0(x, w1, b1):
    b = 2000

    def body(x_ref, w_ref, b_ref, o_ref):
        o_ref[...] = (jnp.dot(x_ref[...], w_ref[...],
                              preferred_element_type=_f32) + b_ref[...])

    return pl.pallas_call(
        body,
        grid=(_N // b,),
        in_specs=[pl.BlockSpec((b, 128), lambda i: (i, 0)),
                  pl.BlockSpec((128, _W), lambda i: (0, 0)),
                  pl.BlockSpec((1, _W), lambda i: (0, 0))],
        out_specs=pl.BlockSpec((b, _W), lambda i: (i, 0)),
        out_shape=jax.ShapeDtypeStruct((_N, _W), _f32),
    )(x, w1, b1)


def _stats_finalize(o_ref, nblk, i):
    @pl.when(i == nblk - 1)
    def _():
        s = o_ref[0:1, :]
        q = o_ref[1:2, :]
        m = s * (1.0 / _E)
        v = q * (1.0 / _E) - m * m
        o_ref[0:1, :] = m
        o_ref[1:2, :] = v


def _tc_stats1(attr, w0, b0, r0, w1, b1, r1):
    b = 1600
    nblk = _E // b

    def body(a_ref, w0r, b0r, r0r, w1r, b1r, r1r, o_ref):
        i = pl.program_id(0)

        @pl.when(i == 0)
        def _():
            o_ref[...] = jnp.zeros((2, 64), _f32)

        kx0 = _ps_scalar(a_ref[...], w0r, b0r, r0r)
        kx1 = _ps_mat(kx0, w1r, b1r, r1r, 64)
        o_ref[0:1, :] += jnp.sum(kx1, axis=0, keepdims=True)
        o_ref[1:2, :] += jnp.sum(kx1 * kx1, axis=0, keepdims=True)
        _stats_finalize(o_ref, nblk, i)

    full = lambda s: pl.BlockSpec(s, lambda i: tuple(0 for _ in s))
    return pl.pallas_call(
        body,
        grid=(nblk,),
        in_specs=[pl.BlockSpec((b, 1), lambda i: (i, 0)),
                  full((1, 256)), full((1, 256)), full((1, 256)),
                  full((64, 256)), full((1, 256)), full((1, 256))],
        out_specs=pl.BlockSpec((2, 64), lambda i: (0, 0)),
        out_shape=jax.ShapeDtypeStruct((2, 64), _f32),
    )(attr, w0, b0, r0, w1, b1, r1)


def _tc_stats2(attr, w0, b0, r0, w1, b1, r1, st1, bng, bnb, w2, b2, r2):
    b = 1600
    nblk = _E // b

    def body(a_ref, w0r, b0r, r0r, w1r, b1r, r1r, st1r, gr, br, w2r, b2r, r2r,
             o_ref):
        i = pl.program_id(0)

        @pl.when(i == 0)
        def _():
            o_ref[...] = jnp.zeros((2, 64), _f32)

        kx0 = _ps_scalar(a_ref[...], w0r, b0r, r0r)
        kx1 = _ps_mat(kx0, w1r, b1r, r1r, 64)
        bn1 = _bn_apply(kx1, st1r, gr, br)
        kx2 = _ps_mat(bn1, w2r, b2r, r2r, 64)
        o_ref[0:1, :] += jnp.sum(kx2, axis=0, keepdims=True)
        o_ref[1:2, :] += jnp.sum(kx2 * kx2, axis=0, keepdims=True)
        _stats_finalize(o_ref, nblk, i)

    full = lambda s: pl.BlockSpec(s, lambda i: tuple(0 for _ in s))
    return pl.pallas_call(
        body,
        grid=(nblk,),
        in_specs=[pl.BlockSpec((b, 1), lambda i: (i, 0)),
                  full((1, 256)), full((1, 256)), full((1, 256)),
                  full((64, 256)), full((1, 256)), full((1, 256)),
                  full((2, 64)), full((1, 64)), full((1, 64)),
                  full((64, 256)), full((1, 256)), full((1, 256))],
        out_specs=pl.BlockSpec((2, 64), lambda i: (0, 0)),
        out_shape=jax.ShapeDtypeStruct((2, 64), _f32),
    )(attr, w0, b0, r0, w1, b1, r1, st1, bng, bnb, w2, b2, r2)


def _tc_weights(attr_pad, w0, b0, r0, w1, b1, r1, st1, bng, bnb,
                w2, b2, r2, st2, wo, bo, ro, dw1, db1, dw2, db2, dw3, db3):
    b = _BM
    nblk = _E_PAD // b

    def body(a_ref, w0r, b0r, r0r, w1r, b1r, r1r, st1r, gr, br,
             w2r, b2r, r2r, st2r, wor, bor, ror, d1r, e1r, d2r, e2r, d3r, e3r,
             wk_ref, ws_ref):
        a = a_ref[...]
        kx0 = _ps_scalar(a, w0r, b0r, r0r)
        kx1 = _ps_mat(kx0, w1r, b1r, r1r, 64)
        bn1 = _bn_apply(kx1, st1r, gr, br)
        kx2 = _ps_mat(bn1, w2r, b2r, r2r, 64)
        bn2 = _bn_apply(kx2, st2r, gr, br)
        wk = _ps_mat(bn2, wor, bor, ror, 256)
        hd = jnp.maximum(a * d1r[...] + e1r[...], 0.0)
        hd = jnp.maximum(
            jnp.dot(hd.astype(jnp.bfloat16), d2r[...],
                    preferred_element_type=_f32) + e2r[...], 0.0)
        wop = jnp.dot(hd.astype(jnp.bfloat16), d3r[...],
                      preferred_element_type=_f32) + e3r[...]
        wk_ref[...] = wk.astype(jnp.bfloat16)
        ws_ref[...] = (wk + wop).astype(jnp.bfloat16)

    full = lambda s: pl.BlockSpec(s, lambda i: tuple(0 for _ in s))
    return pl.pallas_call(
        body,
        grid=(nblk,),
        in_specs=[pl.BlockSpec((b, 1), lambda i: (i, 0)),
                  full((1, 256)), full((1, 256)), full((1, 256)),
                  full((64, 256)), full((1, 256)), full((1, 256)),
                  full((2, 64)), full((1, 64)), full((1, 64)),
                  full((64, 256)), full((1, 256)), full((1, 256)),
                  full((2, 64)),
                  full((64, 1024)), full((1, 1024)), full((1, 1024)),
                  full((1, 128)), full((1, 128)),
                  full((128, 128)), full((1, 128)),
                  full((128, 256)), full((1, 256))],
        out_specs=[pl.BlockSpec((b, 256), lambda i: (i, 0)),
                   pl.BlockSpec((b, 256), lambda i: (i, 0))],
        out_shape=[jax.ShapeDtypeStruct((_E_PAD, 256), jnp.bfloat16),
                   jax.ShapeDtypeStruct((_E_PAD, 256), jnp.bfloat16)],
    )(attr_pad, w0, b0, r0, w1, b1, r1, st1, bng, bnb,
      w2, b2, r2, st2, wo, bo, ro, dw1, db1, dw2, db2, dw3, db3)


def _tc_msg(xjp, xip, wk, wsum, rep, red):
    b = _BM
    bp = b // 8
    nblk = _E_PAD // b

    def body(xj_ref, xi_ref, wk_ref, ws_ref, rep_ref, red_ref, o_ref):
        xp = xj_ref[...]
        ip = xi_ref[...]
        xjv = jnp.concatenate(
            [xp[:, 16 * p:16 * (p + 1)] for p in range(8)], axis=0)
        xiv = jnp.concatenate(
            [ip[:, 16 * p:16 * (p + 1)] for p in range(8)], axis=0)
        xjr = jnp.dot(xjv, rep_ref[...], preferred_element_type=_f32)
        xir = jnp.dot(xiv, rep_ref[...], preferred_element_type=_f32)
        t = (xjr * ws_ref[...].astype(_f32)
             - xir * wk_ref[...].astype(_f32))
        msg = jnp.dot(t, red_ref[...], preferred_element_type=_f32)
        for p in range(8):
            o_ref[:, 16 * p:16 * (p + 1)] = msg[bp * p:bp * (p + 1), :]

    full = lambda s: pl.BlockSpec(s, lambda i: tuple(0 for _ in s))
    return pl.pallas_call(
        body,
        grid=(nblk,),
        in_specs=[pl.BlockSpec((bp, 128), lambda i: (i, 0)),
                  pl.BlockSpec((bp, 128), lambda i: (i, 0)),
                  pl.BlockSpec((b, 256), lambda i: (i, 0)),
                  pl.BlockSpec((b, 256), lambda i: (i, 0)),
                  full((_W, 256)), full((256, _W))],
        out_specs=pl.BlockSpec((bp, 128), lambda i: (i, 0)),
        out_shape=jax.ShapeDtypeStruct((_EP, 128), _f32),
    )(xjp, xip, wk, wsum, rep, red)


def _tc_update(p0, p1, c0, c1, h, root, kb):
    b = 2000

    def body(p0r, p1r, c0r, c1r, hr, rtr, kbr, o_ref):
        agg = (p0r[...] + p1r[...]) / jnp.maximum(c0r[...] + c1r[...], 1.0)
        hv = jnp.dot(hr[...], rtr[...], preferred_element_type=_f32)
        o_ref[...] = jnp.maximum(agg + hv + kbr[...], 0.0)

    full = lambda s: pl.BlockSpec(s, lambda i: tuple(0 for _ in s))
    blk = pl.BlockSpec((b, _W), lambda i: (i, 0))
    return pl.pallas_call(
        body,
        grid=(_N // b,),
        in_specs=[blk, blk, blk, blk, blk, full((_W, _W)), full((1, _W))],
        out_specs=pl.BlockSpec((b, _W), lambda i: (i, 0)),
        out_shape=jax.ShapeDtypeStruct((_N, _W), _f32),
    )(p0, p1, c0, c1, h, root, kb)


def _tc_update_out(p0, p1, c0, c1, h, root, kb, wout, bout):
    b = 2000

    def body(p0r, p1r, c0r, c1r, hr, rtr, kbr, wor, bor, o_ref):
        agg = (p0r[...] + p1r[...]) / jnp.maximum(c0r[...] + c1r[...], 1.0)
        hv = jnp.dot(hr[...], rtr[...], preferred_element_type=_f32)
        hn = jnp.maximum(agg + hv + kbr[...], 0.0)
        o_ref[...] = (jnp.dot(hn, wor[...], preferred_element_type=_f32)
                      + bor[...])

    full = lambda s: pl.BlockSpec(s, lambda i: tuple(0 for _ in s))
    blk = pl.BlockSpec((b, _W), lambda i: (i, 0))
    return pl.pallas_call(
        body,
        grid=(_N // b,),
        in_specs=[blk, blk, blk, blk, blk, full((_W, _W)), full((1, _W)),
                  full((_W, 128)), full((1, 128))],
        out_specs=pl.BlockSpec((b, 128), lambda i: (i, 0)),
        out_shape=jax.ShapeDtypeStruct((_N, 128), _f32),
    )(p0, p1, c0, c1, h, root, kb, wout, bout)


@functools.cache
def _sc_kernels():
    mesh = plsc.VectorSubcoreMesh(core_axis_name="c", subcore_axis_name="s")
    params = pltpu.CompilerParams(use_tc_tiling_on_sc=False)

    @functools.partial(
        pl.kernel,
        out_type=[jax.ShapeDtypeStruct((_E_PAD, _W), _f32),
                  jax.ShapeDtypeStruct((_E_PAD, _W), _f32)],
        mesh=mesh,
        compiler_params=params,
        scratch_types=[pltpu.VMEM((_NCHUNK, _CH), jnp.int32),
                       pltpu.VMEM((_EW, _W), _f32),
                       pltpu.SemaphoreType.DMA],
    )
    def sc_gather(h_hbm, src_hbm, dst_hbm, xj_hbm, xi_hbm, idx_v, rows_v,
                  sem):
        w = lax.axis_index("s") * _NC + lax.axis_index("c")
        for idx_hbm, out_hbm in ((src_hbm, xj_hbm), (dst_hbm, xi_hbm)):
            pltpu.sync_copy(idx_hbm.at[w], idx_v)

            def fire(j, carry):
                pltpu.async_copy(h_hbm.at[idx_v.at[j]],
                                 rows_v.at[pl.ds(j * _CH, _CH)], sem)
                return carry

            def drain(j, carry):
                pltpu.make_async_copy(
                    h_hbm.at[idx_v.at[j]],
                    rows_v.at[pl.ds(j * _CH, _CH)], sem).wait()
                return carry

            lax.fori_loop(0, _NCHUNK, fire, 0)
            lax.fori_loop(0, _NCHUNK, drain, 0)
            pltpu.sync_copy(rows_v, out_hbm.at[pl.ds(w * _EW, _EW)])

    @functools.partial(
        pl.kernel,
        out_type=jax.ShapeDtypeStruct((_NC * _ND, _W), _f32),
        mesh=mesh,
        compiler_params=params,
        scratch_types=[pltpu.VMEM((_NCHUNK, _CH), jnp.int32),
                       pltpu.VMEM((_EW, _W), _f32),
                       pltpu.VMEM_SHARED((_ND, _W), _f32),
                       pltpu.SemaphoreType.DMA],
    )
    def sc_scatter(msg_hbm, dst_hbm, zeros_hbm, out_hbm, idx_v, msg_v,
                   agg_sh, sem):
        cid = lax.axis_index("c")
        sid = lax.axis_index("s")
        w = sid * _NC + cid
        pltpu.sync_copy(zeros_hbm.at[pl.ds(sid * _STRIPE, _STRIPE)],
                        agg_sh.at[pl.ds(sid * _STRIPE, _STRIPE)])
        plsc.subcore_barrier()
        pltpu.sync_copy(dst_hbm.at[w], idx_v)
        pltpu.sync_copy(msg_hbm.at[pl.ds(w * _EW, _EW)], msg_v)

        def fire(j, carry):
            pltpu.async_copy(msg_v.at[pl.ds(j * _CH, _CH)],
                             agg_sh.at[idx_v.at[j]], sem, add=True)
            return carry

        def drain(j, carry):
            pltpu.make_async_copy(msg_v.at[pl.ds(j * _CH, _CH)],
                                  agg_sh.at[idx_v.at[j]], sem).wait()
            return carry

        lax.fori_loop(0, _NCHUNK, fire, 0)
        lax.fori_loop(0, _NCHUNK, drain, 0)
        plsc.subcore_barrier()
        pltpu.sync_copy(agg_sh.at[pl.ds(sid * _STRIPE, _STRIPE)],
                        out_hbm.at[pl.ds(cid * _ND + sid * _STRIPE,
                                         _STRIPE)])

    @functools.partial(
        pl.kernel,
        out_type=jax.ShapeDtypeStruct((_NC * _ND, _W), _f32),
        mesh=mesh,
        compiler_params=params,
        scratch_types=[pltpu.VMEM((_NCHUNK, _CH), jnp.int32),
                       pltpu.VMEM((_CH, _W), _f32),
                       pltpu.VMEM_SHARED((_ND, _W), _f32),
                       pltpu.SemaphoreType.DMA],
    )
    def sc_count(dst_hbm, zeros_hbm, ones_hbm, out_hbm, idx_v, ones_v,
                 cnt_sh, sem):
        cid = lax.axis_index("c")
        sid = lax.axis_index("s")
        w = sid * _NC + cid
        pltpu.sync_copy(zeros_hbm.at[pl.ds(sid * _STRIPE, _STRIPE)],
                        cnt_sh.at[pl.ds(sid * _STRIPE, _STRIPE)])
        pltpu.sync_copy(ones_hbm, ones_v)
        plsc.subcore_barrier()
        pltpu.sync_copy(dst_hbm.at[w], idx_v)

        def fire(j, carry):
            pltpu.async_copy(ones_v, cnt_sh.at[idx_v.at[j]], sem, add=True)
            return carry

        def drain(j, carry):
            pltpu.make_async_copy(ones_v, cnt_sh.at[idx_v.at[j]], sem).wait()
            return carry

        lax.fori_loop(0, _NCHUNK, fire, 0)
        lax.fori_loop(0, _NCHUNK, drain, 0)
        plsc.subcore_barrier()
        pltpu.sync_copy(cnt_sh.at[pl.ds(sid * _STRIPE, _STRIPE)],
                        out_hbm.at[pl.ds(cid * _ND + sid * _STRIPE,
                                         _STRIPE)])

    return sc_gather, sc_scatter, sc_count


def _sc_gather(h, src3, dst3):
    return _sc_kernels()[0](h, src3, dst3)


def _sc_scatter(msg, dst3, zeros_nd):
    return _sc_kernels()[1](msg, dst3, zeros_nd)


def _sc_count(dst3, zeros_nd, ones_ch):
    return _sc_kernels()[2](dst3, zeros_nd, ones_ch)


def kernel(x, edge_index, edge_attr, W1, b1, Wout, bout, root_param, kbias,
           ps0_W, ps0_b, ps0_r, ps1_W, ps1_b, ps1_r, ps2_W, ps2_b, ps2_r,
           psout_W, psout_b, psout_r, bn_g, bn_b, dW1, db1, dW2, db2, dW3,
           db3):
    pad = _E_PAD - _E
    lidx = jnp.arange(_E_PAD, dtype=jnp.int32)
    blk = lidx // _BM
    loc = lidx % _BM
    perm = blk * _BM + (loc % 8) * 128 + loc // 8
    src_pad = jnp.concatenate([edge_index[0], jnp.zeros((pad,), jnp.int32)])
    dst_pad = jnp.concatenate(
        [edge_index[1], jnp.full((pad,), _DUMMY, jnp.int32)])
    src3 = src_pad[perm].reshape(_NW, _NCHUNK, _CH)
    dst3 = dst_pad[perm].reshape(_NW, _NCHUNK, _CH)
    attr_pad = jnp.concatenate([edge_attr, jnp.zeros((pad, 1), _f32)])
    zeros_nd = jnp.zeros((_ND, _W), _f32)
    ones_ch = jnp.ones((_CH, _W), _f32)

    eye = jnp.eye(_W, dtype=_f32)
    rep = jnp.kron(eye, jnp.ones((1, _W), _f32))
    red = jnp.kron(jnp.ones((_W, 1), _f32), eye)

    b1r = b1.reshape(1, _W)
    kbr = kbias.reshape(1, _W)
    boutr = bout.reshape(1, 128)
    bngr = bn_g.reshape(1, 64)
    bnbr = bn_b.reshape(1, 64)
    bf16 = jnp.bfloat16
    w0 = ps0_W.reshape(1, 256)
    b0 = ps0_b.reshape(1, 256)
    r0 = ps0_r.reshape(1, 256)
    w1c = jnp.transpose(ps1_W, (1, 0, 2)).reshape(64, 256).astype(bf16)
    b1c = ps1_b.reshape(1, 256)
    r1c = ps1_r.reshape(1, 256)
    w2c = jnp.transpose(ps2_W, (1, 0, 2)).reshape(64, 256).astype(bf16)
    b2c = ps2_b.reshape(1, 256)
    r2c = ps2_r.reshape(1, 256)
    woc = jnp.transpose(psout_W, (1, 0, 2)).reshape(64, 1024).astype(bf16)
    boc = psout_b.reshape(1, 1024)
    roc = psout_r.reshape(1, 1024)
    d1r = dW1.reshape(1, 128)
    e1r = db1.reshape(1, 128)
    e2r = db2.reshape(1, 128)
    e3r = db3.reshape(1, 256)
    dw2b = dW2.astype(bf16)
    dw3b = dW3.astype(bf16)

    h0 = _tc_h0(x, W1, b1r)
    cnts = _sc_count(dst3, zeros_nd, ones_ch)
    st1 = _tc_stats1(edge_attr, w0, b0, r0, w1c, b1c, r1c)
    st2 = _tc_stats2(edge_attr, w0, b0, r0, w1c, b1c, r1c,
                     st1, bngr, bnbr, w2c, b2c, r2c)
    wk, wsum = _tc_weights(attr_pad, w0, b0, r0, w1c, b1c, r1c,
                           st1, bngr, bnbr, w2c, b2c, r2c, st2,
                           woc, boc, roc,
                           d1r, e1r, dw2b, e2r, dw3b, e3r)
    c0 = cnts[:_N]
    c1 = cnts[_ND:_ND + _N]

    h = h0
    for layer in range(2):
        xj, xi = _sc_gather(h, src3, dst3)
        msg = _tc_msg(xj.reshape(_EP, 128), xi.reshape(_EP, 128),
                      wk, wsum, rep, red)
        parts = _sc_scatter(msg.reshape(_E_PAD, _W), dst3, zeros_nd)
        p0 = parts[:_N]
        p1 = parts[_ND:_ND + _N]
        if layer == 0:
            h = _tc_update(p0, p1, c0, c1, h, root_param, kbr)
        else:
            h = _tc_update_out(p0, p1, c0, c1, h, root_param, kbr, Wout,
                               boutr)
    return h


# speedup vs baseline: 3.4438x; 1.0874x over previous
import functools

import jax
import jax.numpy as jnp
from jax import lax
from jax.experimental import pallas as pl
from jax.experimental.pallas import tpu as pltpu
from jax.experimental.pallas import tpu_sc as plsc

_N = 10000
_E = 160000
_W = 16
_NC = 2
_NS = 16
_NW = _NC * _NS
_CH = 128
_NCHUNK = 40
_EW = _CH * _NCHUNK
_E_PAD = _EW * _NW
_EP = _E_PAD // 8
_ND = 10240
_STRIPE = _ND // _NS
_DUMMY = _N
_BM = 2048

_f32 = jnp.float32


def _leaky(x):
    return jnp.where(x >= 0, x, 0.1 * x)


def _ps_combine(y, r_ref, k):
    y0 = y[:, 0:k]
    y1 = _leaky(y[:, k:2 * k])
    y2 = _leaky(y[:, 2 * k:3 * k])
    y3 = _leaky(y[:, 3 * k:4 * k])
    return (r_ref[:, 0:k] * y0 + r_ref[:, k:2 * k] * y1
            + r_ref[:, 2 * k:3 * k] * (y2 * y2)
            + r_ref[:, 3 * k:4 * k] * (y3 * y3 * y3))


def _ps_scalar(a, w_ref, b_ref, r_ref):
    y = a * w_ref[...] + b_ref[...]
    return _ps_combine(y, r_ref, 64)


def _ps_mat(x, w_ref, b_ref, r_ref, k):
    y = jnp.dot(x.astype(jnp.bfloat16), w_ref[...],
                preferred_element_type=_f32) + b_ref[...]
    return _ps_combine(y, r_ref, k)


def _attr_col(a_ref, b):
    rows = b // 128
    eye = jnp.eye(rows, dtype=_f32)
    at = lax.dot_general(a_ref[...], eye, (((0,), (0,)), ((), ())),
                         precision=lax.Precision.HIGHEST,
                         preferred_element_type=_f32)
    return jnp.concatenate(
        [at[:, r:r + 1] for r in range(rows)], axis=0)


def _bn_apply(x, st_ref, g_ref, b_ref):
    return ((x - st_ref[0:1, :]) * lax.rsqrt(st_ref[1:2, :] + 1e-5)
            * g_ref[0:1, :] + b_ref[0:1, :])


def _tc_h0(x, w1, b1):
    b = 2000

    def body(x_ref, w_ref, b_ref, o_ref):
        o_ref[...] = (jnp.dot(x_ref[...], w_ref[...],
                              preferred_element_type=_f32) + b_ref[...])

    return pl.pallas_call(
        body,
        grid=(_N // b,),
        in_specs=[pl.BlockSpec((b, 128), lambda i: (i, 0)),
                  pl.BlockSpec((128, _W), lambda i: (0, 0)),
                  pl.BlockSpec((1, _W), lambda i: (0, 0))],
        out_specs=pl.BlockSpec((b, _W), lambda i: (i, 0)),
        out_shape=jax.ShapeDtypeStruct((_N, _W), _f32),
    )(x, w1, b1)


def _stats_finalize(o_ref, nblk, i):
    @pl.when(i == nblk - 1)
    def _():
        s = o_ref[0:1, :]
        q = o_ref[1:2, :]
        m = s * (1.0 / _E)
        v = q * (1.0 / _E) - m * m
        o_ref[0:1, :] = m
        o_ref[1:2, :] = v


def _tc_stats1(attr, w0, b0, r0, w1, b1, r1):
    b = 2048
    nblk = _E_PAD // b

    def body(a_ref, w0r, b0r, r0r, w1r, b1r, r1r, o_ref):
        i = pl.program_id(0)

        @pl.when(i == 0)
        def _():
            o_ref[...] = jnp.zeros((2, 64), _f32)

        kx0 = _ps_scalar(_attr_col(a_ref, b), w0r, b0r, r0r)
        kx1 = _ps_mat(kx0, w1r, b1r, r1r, 64)
        rows = lax.broadcasted_iota(jnp.int32, (b, 64), 0)
        kx1 = jnp.where(rows < _E - i * b, kx1, 0.0)
        o_ref[0:1, :] += jnp.sum(kx1, axis=0, keepdims=True)
        o_ref[1:2, :] += jnp.sum(kx1 * kx1, axis=0, keepdims=True)
        _stats_finalize(o_ref, nblk, i)

    full = lambda s: pl.BlockSpec(s, lambda i: tuple(0 for _ in s))
    return pl.pallas_call(
        body,
        grid=(nblk,),
        in_specs=[pl.BlockSpec((b // 128, 128), lambda i: (i, 0)),
                  full((1, 256)), full((1, 256)), full((1, 256)),
                  full((64, 256)), full((1, 256)), full((1, 256))],
        out_specs=pl.BlockSpec((2, 64), lambda i: (0, 0)),
        out_shape=jax.ShapeDtypeStruct((2, 64), _f32),
    )(attr, w0, b0, r0, w1, b1, r1)


def _tc_stats2(attr, w0, b0, r0, w1, b1, r1, st1, bng, bnb, w2, b2, r2):
    b = 2048
    nblk = _E_PAD // b

    def body(a_ref, w0r, b0r, r0r, w1r, b1r, r1r, st1r, gr, br, w2r, b2r, r2r,
             o_ref):
        i = pl.program_id(0)

        @pl.when(i == 0)
        def _():
            o_ref[...] = jnp.zeros((2, 64), _f32)

        kx0 = _ps_scalar(_attr_col(a_ref, b), w0r, b0r, r0r)
        kx1 = _ps_mat(kx0, w1r, b1r, r1r, 64)
        bn1 = _bn_apply(kx1, st1r, gr, br)
        kx2 = _ps_mat(bn1, w2r, b2r, r2r, 64)
        rows = lax.broadcasted_iota(jnp.int32, (b, 64), 0)
        kx2 = jnp.where(rows < _E - i * b, kx2, 0.0)
        o_ref[0:1, :] += jnp.sum(kx2, axis=0, keepdims=True)
        o_ref[1:2, :] += jnp.sum(kx2 * kx2, axis=0, keepdims=True)
        _stats_finalize(o_ref, nblk, i)

    full = lambda s: pl.BlockSpec(s, lambda i: tuple(0 for _ in s))
    return pl.pallas_call(
        body,
        grid=(nblk,),
        in_specs=[pl.BlockSpec((b // 128, 128), lambda i: (i, 0)),
                  full((1, 256)), full((1, 256)), full((1, 256)),
                  full((64, 256)), full((1, 256)), full((1, 256)),
                  full((2, 64)), full((1, 64)), full((1, 64)),
                  full((64, 256)), full((1, 256)), full((1, 256))],
        out_specs=pl.BlockSpec((2, 64), lambda i: (0, 0)),
        out_shape=jax.ShapeDtypeStruct((2, 64), _f32),
    )(attr, w0, b0, r0, w1, b1, r1, st1, bng, bnb, w2, b2, r2)


def _tc_weights(attr_pad, w0, b0, r0, w1, b1, r1, st1, bng, bnb,
                w2, b2, r2, st2, wo, bo, ro, dw1, db1, dw2, db2, dw3, db3):
    b = _BM
    nblk = _E_PAD // b

    def body(a_ref, w0r, b0r, r0r, w1r, b1r, r1r, st1r, gr, br,
             w2r, b2r, r2r, st2r, wor, bor, ror, d1r, e1r, d2r, e2r, d3r, e3r,
             wk_ref, ws_ref):
        a = _attr_col(a_ref, b)
        kx0 = _ps_scalar(a, w0r, b0r, r0r)
        kx1 = _ps_mat(kx0, w1r, b1r, r1r, 64)
        bn1 = _bn_apply(kx1, st1r, gr, br)
        kx2 = _ps_mat(bn1, w2r, b2r, r2r, 64)
        bn2 = _bn_apply(kx2, st2r, gr, br)
        wk = _ps_mat(bn2, wor, bor, ror, 256)
        hd = jnp.maximum(a * d1r[...] + e1r[...], 0.0)
        hd = jnp.maximum(
            jnp.dot(hd.astype(jnp.bfloat16), d2r[...],
                    preferred_element_type=_f32) + e2r[...], 0.0)
        wop = jnp.dot(hd.astype(jnp.bfloat16), d3r[...],
                      preferred_element_type=_f32) + e3r[...]
        wk_ref[...] = wk.astype(jnp.bfloat16)
        ws_ref[...] = (wk + wop).astype(jnp.bfloat16)

    full = lambda s: pl.BlockSpec(s, lambda i: tuple(0 for _ in s))
    return pl.pallas_call(
        body,
        grid=(nblk,),
        in_specs=[pl.BlockSpec((b // 128, 128), lambda i: (i, 0)),
                  full((1, 256)), full((1, 256)), full((1, 256)),
                  full((64, 256)), full((1, 256)), full((1, 256)),
                  full((2, 64)), full((1, 64)), full((1, 64)),
                  full((64, 256)), full((1, 256)), full((1, 256)),
                  full((2, 64)),
                  full((64, 1024)), full((1, 1024)), full((1, 1024)),
                  full((1, 128)), full((1, 128)),
                  full((128, 128)), full((1, 128)),
                  full((128, 256)), full((1, 256))],
        out_specs=[pl.BlockSpec((b, 256), lambda i: (i, 0)),
                   pl.BlockSpec((b, 256), lambda i: (i, 0))],
        out_shape=[jax.ShapeDtypeStruct((_E_PAD, 256), jnp.bfloat16),
                   jax.ShapeDtypeStruct((_E_PAD, 256), jnp.bfloat16)],
    )(attr_pad, w0, b0, r0, w1, b1, r1, st1, bng, bnb,
      w2, b2, r2, st2, wo, bo, ro, dw1, db1, dw2, db2, dw3, db3)


def _tc_msg(xjp, xip, wk, wsum, rep, red):
    b = _BM
    bp = b // 8
    nblk = _E_PAD // b

    def body(xj_ref, xi_ref, wk_ref, ws_ref, rep_ref, red_ref, o_ref):
        xp = xj_ref[...]
        ip = xi_ref[...]
        xjv = jnp.concatenate(
            [xp[:, 16 * p:16 * (p + 1)] for p in range(8)], axis=0)
        xiv = jnp.concatenate(
            [ip[:, 16 * p:16 * (p + 1)] for p in range(8)], axis=0)
        xjr = jnp.dot(xjv, rep_ref[...], preferred_element_type=_f32)
        xir = jnp.dot(xiv, rep_ref[...], preferred_element_type=_f32)
        t = (xjr * ws_ref[...].astype(_f32)
             - xir * wk_ref[...].astype(_f32))
        msg = jnp.dot(t, red_ref[...], preferred_element_type=_f32)
        for p in range(8):
            o_ref[:, 16 * p:16 * (p + 1)] = msg[bp * p:bp * (p + 1), :]

    full = lambda s: pl.BlockSpec(s, lambda i: tuple(0 for _ in s))
    return pl.pallas_call(
        body,
        grid=(nblk,),
        in_specs=[pl.BlockSpec((bp, 128), lambda i: (i, 0)),
                  pl.BlockSpec((bp, 128), lambda i: (i, 0)),
                  pl.BlockSpec((b, 256), lambda i: (i, 0)),
                  pl.BlockSpec((b, 256), lambda i: (i, 0)),
                  full((_W, 256)), full((256, _W))],
        out_specs=pl.BlockSpec((bp, 128), lambda i: (i, 0)),
        out_shape=jax.ShapeDtypeStruct((_EP, 128), _f32),
    )(xjp, xip, wk, wsum, rep, red)


def _tc_update(p0, p1, c0, c1, h, root, kb):
    b = 2000

    def body(p0r, p1r, c0r, c1r, hr, rtr, kbr, o_ref):
        agg = (p0r[...] + p1r[...]) / jnp.maximum(c0r[...] + c1r[...], 1.0)
        hv = jnp.dot(hr[...], rtr[...], preferred_element_type=_f32)
        o_ref[...] = jnp.maximum(agg + hv + kbr[...], 0.0)

    full = lambda s: pl.BlockSpec(s, lambda i: tuple(0 for _ in s))
    blk = pl.BlockSpec((b, _W), lambda i: (i, 0))
    return pl.pallas_call(
        body,
        grid=(_N // b,),
        in_specs=[blk, blk, blk, blk, blk, full((_W, _W)), full((1, _W))],
        out_specs=pl.BlockSpec((b, _W), lambda i: (i, 0)),
        out_shape=jax.ShapeDtypeStruct((_N, _W), _f32),
    )(p0, p1, c0, c1, h, root, kb)


def _tc_update_out(p0, p1, c0, c1, h, root, kb, wout, bout):
    b = 2000

    def body(p0r, p1r, c0r, c1r, hr, rtr, kbr, wor, bor, o_ref):
        agg = (p0r[...] + p1r[...]) / jnp.maximum(c0r[...] + c1r[...], 1.0)
        hv = jnp.dot(hr[...], rtr[...], preferred_element_type=_f32)
        hn = jnp.maximum(agg + hv + kbr[...], 0.0)
        o_ref[...] = (jnp.dot(hn, wor[...], preferred_element_type=_f32)
                      + bor[...])

    full = lambda s: pl.BlockSpec(s, lambda i: tuple(0 for _ in s))
    blk = pl.BlockSpec((b, _W), lambda i: (i, 0))
    return pl.pallas_call(
        body,
        grid=(_N // b,),
        in_specs=[blk, blk, blk, blk, blk, full((_W, _W)), full((1, _W)),
                  full((_W, 128)), full((1, 128))],
        out_specs=pl.BlockSpec((b, 128), lambda i: (i, 0)),
        out_shape=jax.ShapeDtypeStruct((_N, 128), _f32),
    )(p0, p1, c0, c1, h, root, kb, wout, bout)


@functools.cache
def _sc_kernels():
    mesh = plsc.VectorSubcoreMesh(core_axis_name="c", subcore_axis_name="s")
    params = pltpu.CompilerParams(use_tc_tiling_on_sc=False)

    @functools.partial(
        pl.kernel,
        out_type=[jax.ShapeDtypeStruct((_E_PAD, _W), _f32),
                  jax.ShapeDtypeStruct((_E_PAD, _W), _f32)],
        mesh=mesh,
        compiler_params=params,
        scratch_types=[pltpu.VMEM((_NCHUNK, _CH), jnp.int32),
                       pltpu.VMEM((_EW, _W), _f32),
                       pltpu.SemaphoreType.DMA],
    )
    def sc_gather(h_hbm, src_hbm, dst_hbm, xj_hbm, xi_hbm, idx_v, rows_v,
                  sem):
        w = lax.axis_index("s") * _NC + lax.axis_index("c")
        for idx_hbm, out_hbm in ((src_hbm, xj_hbm), (dst_hbm, xi_hbm)):
            pltpu.sync_copy(idx_hbm.at[w], idx_v)

            def fire(j, carry):
                pltpu.async_copy(h_hbm.at[idx_v.at[j]],
                                 rows_v.at[pl.ds(j * _CH, _CH)], sem)
                return carry

            def drain(j, carry):
                pltpu.make_async_copy(
                    h_hbm.at[idx_v.at[j]],
                    rows_v.at[pl.ds(j * _CH, _CH)], sem).wait()
                return carry

            lax.fori_loop(0, _NCHUNK, fire, 0)
            lax.fori_loop(0, _NCHUNK, drain, 0)
            pltpu.sync_copy(rows_v, out_hbm.at[pl.ds(w * _EW, _EW)])

    @functools.partial(
        pl.kernel,
        out_type=jax.ShapeDtypeStruct((_NC * _ND, _W), _f32),
        mesh=mesh,
        compiler_params=params,
        scratch_types=[pltpu.VMEM((_NCHUNK, _CH), jnp.int32),
                       pltpu.VMEM((_EW, _W), _f32),
                       pltpu.VMEM_SHARED((_ND, _W), _f32),
                       pltpu.SemaphoreType.DMA],
    )
    def sc_scatter(msg_hbm, dst_hbm, zeros_hbm, out_hbm, idx_v, msg_v,
                   agg_sh, sem):
        cid = lax.axis_index("c")
        sid = lax.axis_index("s")
        w = sid * _NC + cid
        pltpu.sync_copy(zeros_hbm.at[pl.ds(sid * _STRIPE, _STRIPE)],
                        agg_sh.at[pl.ds(sid * _STRIPE, _STRIPE)])
        plsc.subcore_barrier()
        pltpu.sync_copy(dst_hbm.at[w], idx_v)
        pltpu.sync_copy(msg_hbm.at[pl.ds(w * _EW, _EW)], msg_v)

        def fire(j, carry):
            pltpu.async_copy(msg_v.at[pl.ds(j * _CH, _CH)],
                             agg_sh.at[idx_v.at[j]], sem, add=True)
            return carry

        def drain(j, carry):
            pltpu.make_async_copy(msg_v.at[pl.ds(j * _CH, _CH)],
                                  agg_sh.at[idx_v.at[j]], sem).wait()
            return carry

        lax.fori_loop(0, _NCHUNK, fire, 0)
        lax.fori_loop(0, _NCHUNK, drain, 0)
        plsc.subcore_barrier()
        pltpu.sync_copy(agg_sh.at[pl.ds(sid * _STRIPE, _STRIPE)],
                        out_hbm.at[pl.ds(cid * _ND + sid * _STRIPE,
                                         _STRIPE)])

    @functools.partial(
        pl.kernel,
        out_type=jax.ShapeDtypeStruct((_NC * _ND, _W), _f32),
        mesh=mesh,
        compiler_params=params,
        scratch_types=[pltpu.VMEM((_NCHUNK, _CH), jnp.int32),
                       pltpu.VMEM((_CH, _W), _f32),
                       pltpu.VMEM_SHARED((_ND, _W), _f32),
                       pltpu.SemaphoreType.DMA],
    )
    def sc_count(dst_hbm, zeros_hbm, ones_hbm, out_hbm, idx_v, ones_v,
                 cnt_sh, sem):
        cid = lax.axis_index("c")
        sid = lax.axis_index("s")
        w = sid * _NC + cid
        pltpu.sync_copy(zeros_hbm.at[pl.ds(sid * _STRIPE, _STRIPE)],
                        cnt_sh.at[pl.ds(sid * _STRIPE, _STRIPE)])
        pltpu.sync_copy(ones_hbm, ones_v)
        plsc.subcore_barrier()
        pltpu.sync_copy(dst_hbm.at[w], idx_v)

        def fire(j, carry):
            pltpu.async_copy(ones_v, cnt_sh.at[idx_v.at[j]], sem, add=True)
            return carry

        def drain(j, carry):
            pltpu.make_async_copy(ones_v, cnt_sh.at[idx_v.at[j]], sem).wait()
            return carry

        lax.fori_loop(0, _NCHUNK, fire, 0)
        lax.fori_loop(0, _NCHUNK, drain, 0)
        plsc.subcore_barrier()
        pltpu.sync_copy(cnt_sh.at[pl.ds(sid * _STRIPE, _STRIPE)],
                        out_hbm.at[pl.ds(cid * _ND + sid * _STRIPE,
                                         _STRIPE)])

    return sc_gather, sc_scatter, sc_count


def _sc_gather(h, src3, dst3):
    return _sc_kernels()[0](h, src3, dst3)


def _sc_scatter(msg, dst3, zeros_nd):
    return _sc_kernels()[1](msg, dst3, zeros_nd)


def _sc_count(dst3, zeros_nd, ones_ch):
    return _sc_kernels()[2](dst3, zeros_nd, ones_ch)


def kernel(x, edge_index, edge_attr, W1, b1, Wout, bout, root_param, kbias,
           ps0_W, ps0_b, ps0_r, ps1_W, ps1_b, ps1_r, ps2_W, ps2_b, ps2_r,
           psout_W, psout_b, psout_r, bn_g, bn_b, dW1, db1, dW2, db2, dW3,
           db3):
    pad = _E_PAD - _E
    lidx = jnp.arange(_E_PAD, dtype=jnp.int32)
    blk = lidx // _BM
    loc = lidx % _BM
    perm = blk * _BM + (loc % 8) * (_BM // 8) + loc // 8
    src_pad = jnp.concatenate([edge_index[0], jnp.zeros((pad,), jnp.int32)])
    dst_pad = jnp.concatenate(
        [edge_index[1], jnp.full((pad,), _DUMMY, jnp.int32)])
    src3 = src_pad[perm].reshape(_NW, _NCHUNK, _CH)
    dst3 = dst_pad[perm].reshape(_NW, _NCHUNK, _CH)
    attr_dense = jnp.concatenate(
        [edge_attr.reshape(_E), jnp.zeros((pad,), _f32)]
    ).reshape(_E_PAD // 128, 128)
    zeros_nd = jnp.zeros((_ND, _W), _f32)
    ones_ch = jnp.ones((_CH, _W), _f32)

    eye = jnp.eye(_W, dtype=_f32)
    rep = jnp.kron(eye, jnp.ones((1, _W), _f32))
    red = jnp.kron(jnp.ones((_W, 1), _f32), eye)

    b1r = b1.reshape(1, _W)
    kbr = kbias.reshape(1, _W)
    boutr = bout.reshape(1, 128)
    bngr = bn_g.reshape(1, 64)
    bnbr = bn_b.reshape(1, 64)
    bf16 = jnp.bfloat16
    w0 = ps0_W.reshape(1, 256)
    b0 = ps0_b.reshape(1, 256)
    r0 = ps0_r.reshape(1, 256)
    w1c = jnp.transpose(ps1_W, (1, 0, 2)).reshape(64, 256).astype(bf16)
    b1c = ps1_b.reshape(1, 256)
    r1c = ps1_r.reshape(1, 256)
    w2c = jnp.transpose(ps2_W, (1, 0, 2)).reshape(64, 256).astype(bf16)
    b2c = ps2_b.reshape(1, 256)
    r2c = ps2_r.reshape(1, 256)
    woc = jnp.transpose(psout_W, (1, 0, 2)).reshape(64, 1024).astype(bf16)
    boc = psout_b.reshape(1, 1024)
    roc = psout_r.reshape(1, 1024)
    d1r = dW1.reshape(1, 128)
    e1r = db1.reshape(1, 128)
    e2r = db2.reshape(1, 128)
    e3r = db3.reshape(1, 256)
    dw2b = dW2.astype(bf16)
    dw3b = dW3.astype(bf16)

    h0 = _tc_h0(x, W1, b1r)
    cnts = _sc_count(dst3, zeros_nd, ones_ch)
    st1 = _tc_stats1(attr_dense, w0, b0, r0, w1c, b1c, r1c)
    st2 = _tc_stats2(attr_dense, w0, b0, r0, w1c, b1c, r1c,
                     st1, bngr, bnbr, w2c, b2c, r2c)
    wk, wsum = _tc_weights(attr_dense, w0, b0, r0, w1c, b1c, r1c,
                           st1, bngr, bnbr, w2c, b2c, r2c, st2,
                           woc, boc, roc,
                           d1r, e1r, dw2b, e2r, dw3b, e3r)
    c0 = cnts[:_N]
    c1 = cnts[_ND:_ND + _N]

    h = h0
    for layer in range(2):
        xj, xi = _sc_gather(h, src3, dst3)
        msg = _tc_msg(xj.reshape(_EP, 128), xi.reshape(_EP, 128),
                      wk, wsum, rep, red)
        parts = _sc_scatter(msg.reshape(_E_PAD, _W), dst3, zeros_nd)
        p0 = parts[:_N]
        p1 = parts[_ND:_ND + _N]
        if layer == 0:
            h = _tc_update(p0, p1, c0, c1, h, root_param, kbr)
        else:
            h = _tc_update_out(p0, p1, c0, c1, h, root_param, kbr, Wout,
                               boutr)
    return h


# speedup vs baseline: 4.5066x; 1.3086x over previous
import functools

import jax
import jax.numpy as jnp
from jax import lax
from jax.experimental import pallas as pl
from jax.experimental.pallas import tpu as pltpu
from jax.experimental.pallas import tpu_sc as plsc

_N = 10000
_E = 160000
_W = 16
_NC = 2
_NS = 16
_NW = _NC * _NS
_CH = 128
_NCHUNK = 40
_EW = _CH * _NCHUNK
_E_PAD = _EW * _NW
_EP = _E_PAD // 8
_ND = 10240
_STRIPE = _ND // _NS
_DUMMY = _N
_BM = 2048

_f32 = jnp.float32


def _leaky(x):
    return jnp.where(x >= 0, x, 0.1 * x)


def _ps_combine(y, r_ref, k):
    y0 = y[:, 0:k]
    y1 = _leaky(y[:, k:2 * k])
    y2 = _leaky(y[:, 2 * k:3 * k])
    y3 = _leaky(y[:, 3 * k:4 * k])
    return (r_ref[:, 0:k] * y0 + r_ref[:, k:2 * k] * y1
            + r_ref[:, 2 * k:3 * k] * (y2 * y2)
            + r_ref[:, 3 * k:4 * k] * (y3 * y3 * y3))


def _ps_scalar(a, w_ref, b_ref, r_ref):
    y = a * w_ref[...] + b_ref[...]
    return _ps_combine(y, r_ref, 64)


def _ps_mat(x, w_ref, b_ref, r_ref, k):
    y = jnp.dot(x.astype(jnp.bfloat16), w_ref[...],
                preferred_element_type=_f32) + b_ref[...]
    return _ps_combine(y, r_ref, k)


def _attr_col(a_ref, b):
    rows = b // 128
    eye = jnp.eye(rows, dtype=_f32)
    at = lax.dot_general(a_ref[...], eye, (((0,), (0,)), ((), ())),
                         precision=lax.Precision.HIGHEST,
                         preferred_element_type=_f32)
    return jnp.concatenate(
        [at[:, r:r + 1] for r in range(rows)], axis=0)


def _bn_apply(x, st_ref, g_ref, b_ref):
    return ((x - st_ref[0:1, :]) * lax.rsqrt(st_ref[1:2, :] + 1e-5)
            * g_ref[0:1, :] + b_ref[0:1, :])


def _tc_h0(x, w1, b1):
    b = 2000

    def body(x_ref, w_ref, b_ref, o_ref):
        o_ref[...] = (jnp.dot(x_ref[...], w_ref[...],
                              preferred_element_type=_f32) + b_ref[...])

    return pl.pallas_call(
        body,
        grid=(_N // b,),
        in_specs=[pl.BlockSpec((b, 128), lambda i: (i, 0)),
                  pl.BlockSpec((128, _W), lambda i: (0, 0)),
                  pl.BlockSpec((1, _W), lambda i: (0, 0))],
        out_specs=pl.BlockSpec((b, _W), lambda i: (i, 0)),
        out_shape=jax.ShapeDtypeStruct((_N, _W), _f32),
    )(x, w1, b1)


def _stats_finalize(o_ref, nblk, i):
    @pl.when(i == nblk - 1)
    def _():
        s = o_ref[0:1, :]
        q = o_ref[1:2, :]
        m = s * (1.0 / _E)
        v = q * (1.0 / _E) - m * m
        o_ref[0:1, :] = m
        o_ref[1:2, :] = v


def _tc_stats1(attr, w0, b0, r0, w1, b1, r1):
    b = 2048
    nblk = _E_PAD // b

    def body(a_ref, w0r, b0r, r0r, w1r, b1r, r1r, o_ref, k_ref):
        i = pl.program_id(0)

        @pl.when(i == 0)
        def _():
            o_ref[...] = jnp.zeros((2, 64), _f32)

        kx0 = _ps_scalar(_attr_col(a_ref, b), w0r, b0r, r0r)
        kx1 = _ps_mat(kx0, w1r, b1r, r1r, 64)
        k_ref[...] = kx1
        rows = lax.broadcasted_iota(jnp.int32, (b, 64), 0)
        kx1 = jnp.where(rows < _E - i * b, kx1, 0.0)
        o_ref[0:1, :] += jnp.sum(kx1, axis=0, keepdims=True)
        o_ref[1:2, :] += jnp.sum(kx1 * kx1, axis=0, keepdims=True)
        _stats_finalize(o_ref, nblk, i)

    full = lambda s: pl.BlockSpec(s, lambda i: tuple(0 for _ in s))
    return pl.pallas_call(
        body,
        grid=(nblk,),
        in_specs=[pl.BlockSpec((b // 128, 128), lambda i: (i, 0)),
                  full((1, 256)), full((1, 256)), full((1, 256)),
                  full((64, 256)), full((1, 256)), full((1, 256))],
        out_specs=[pl.BlockSpec((2, 64), lambda i: (0, 0)),
                   pl.BlockSpec((b, 64), lambda i: (i, 0))],
        out_shape=[jax.ShapeDtypeStruct((2, 64), _f32),
                   jax.ShapeDtypeStruct((_E_PAD, 64), _f32)],
    )(attr, w0, b0, r0, w1, b1, r1)


def _tc_stats2(kx1a, st1, bng, bnb, w2, b2, r2):
    b = 2048
    nblk = _E_PAD // b

    def body(k1_ref, st1r, gr, br, w2r, b2r, r2r, o_ref, k_ref):
        i = pl.program_id(0)

        @pl.when(i == 0)
        def _():
            o_ref[...] = jnp.zeros((2, 64), _f32)

        bn1 = _bn_apply(k1_ref[...], st1r, gr, br)
        kx2 = _ps_mat(bn1, w2r, b2r, r2r, 64)
        k_ref[...] = kx2
        rows = lax.broadcasted_iota(jnp.int32, (b, 64), 0)
        kx2 = jnp.where(rows < _E - i * b, kx2, 0.0)
        o_ref[0:1, :] += jnp.sum(kx2, axis=0, keepdims=True)
        o_ref[1:2, :] += jnp.sum(kx2 * kx2, axis=0, keepdims=True)
        _stats_finalize(o_ref, nblk, i)

    full = lambda s: pl.BlockSpec(s, lambda i: tuple(0 for _ in s))
    return pl.pallas_call(
        body,
        grid=(nblk,),
        in_specs=[pl.BlockSpec((b, 64), lambda i: (i, 0)),
                  full((2, 64)), full((1, 64)), full((1, 64)),
                  full((64, 256)), full((1, 256)), full((1, 256))],
        out_specs=[pl.BlockSpec((2, 64), lambda i: (0, 0)),
                   pl.BlockSpec((b, 64), lambda i: (i, 0))],
        out_shape=[jax.ShapeDtypeStruct((2, 64), _f32),
                   jax.ShapeDtypeStruct((_E_PAD, 64), _f32)],
    )(kx1a, st1, bng, bnb, w2, b2, r2)


def _tc_weights(attr_pad, kx2a, bng, bnb,
                st2, wo, bo, ro, dw1, db1, dw2, db2, dw3, db3):
    b = _BM
    nblk = _E_PAD // b

    def body(a_ref, k2_ref, gr, br,
             st2r, wor, bor, ror, d1r, e1r, d2r, e2r, d3r, e3r,
             wk_ref, ws_ref):
        a = _attr_col(a_ref, b)
        bn2 = _bn_apply(k2_ref[...], st2r, gr, br)
        wk = _ps_mat(bn2, wor, bor, ror, 256)
        hd = jnp.maximum(a * d1r[...] + e1r[...], 0.0)
        hd = jnp.maximum(
            jnp.dot(hd.astype(jnp.bfloat16), d2r[...],
                    preferred_element_type=_f32) + e2r[...], 0.0)
        wop = jnp.dot(hd.astype(jnp.bfloat16), d3r[...],
                      preferred_element_type=_f32) + e3r[...]
        wk_ref[...] = wk.astype(jnp.bfloat16)
        ws_ref[...] = (wk + wop).astype(jnp.bfloat16)

    full = lambda s: pl.BlockSpec(s, lambda i: tuple(0 for _ in s))
    return pl.pallas_call(
        body,
        grid=(nblk,),
        in_specs=[pl.BlockSpec((b // 128, 128), lambda i: (i, 0)),
                  pl.BlockSpec((b, 64), lambda i: (i, 0)),
                  full((1, 64)), full((1, 64)),
                  full((2, 64)),
                  full((64, 1024)), full((1, 1024)), full((1, 1024)),
                  full((1, 128)), full((1, 128)),
                  full((128, 128)), full((1, 128)),
                  full((128, 256)), full((1, 256))],
        out_specs=[pl.BlockSpec((b, 256), lambda i: (i, 0)),
                   pl.BlockSpec((b, 256), lambda i: (i, 0))],
        out_shape=[jax.ShapeDtypeStruct((_E_PAD, 256), jnp.bfloat16),
                   jax.ShapeDtypeStruct((_E_PAD, 256), jnp.bfloat16)],
    )(attr_pad, kx2a, bng, bnb,
      st2, wo, bo, ro, dw1, db1, dw2, db2, dw3, db3)


def _tc_msg(xjp, xip, wk, wsum, rep, red):
    b = _BM
    bp = b // 8
    nblk = _E_PAD // b

    def body(xj_ref, xi_ref, wk_ref, ws_ref, rep_ref, red_ref, o_ref):
        xp = xj_ref[...]
        ip = xi_ref[...]
        xjv = jnp.concatenate(
            [xp[:, 16 * p:16 * (p + 1)] for p in range(8)], axis=0)
        xiv = jnp.concatenate(
            [ip[:, 16 * p:16 * (p + 1)] for p in range(8)], axis=0)
        xjr = jnp.dot(xjv, rep_ref[...], preferred_element_type=_f32)
        xir = jnp.dot(xiv, rep_ref[...], preferred_element_type=_f32)
        t = (xjr * ws_ref[...].astype(_f32)
             - xir * wk_ref[...].astype(_f32))
        msg = jnp.dot(t, red_ref[...], preferred_element_type=_f32)
        for p in range(8):
            o_ref[:, 16 * p:16 * (p + 1)] = msg[bp * p:bp * (p + 1), :]

    full = lambda s: pl.BlockSpec(s, lambda i: tuple(0 for _ in s))
    return pl.pallas_call(
        body,
        grid=(nblk,),
        in_specs=[pl.BlockSpec((bp, 128), lambda i: (i, 0)),
                  pl.BlockSpec((bp, 128), lambda i: (i, 0)),
                  pl.BlockSpec((b, 256), lambda i: (i, 0)),
                  pl.BlockSpec((b, 256), lambda i: (i, 0)),
                  full((_W, 256)), full((256, _W))],
        out_specs=pl.BlockSpec((bp, 128), lambda i: (i, 0)),
        out_shape=jax.ShapeDtypeStruct((_EP, 128), _f32),
    )(xjp, xip, wk, wsum, rep, red)


def _tc_update(p0, p1, c0, c1, h, root, kb):
    b = 2000

    def body(p0r, p1r, c0r, c1r, hr, rtr, kbr, o_ref):
        agg = (p0r[...] + p1r[...]) / jnp.maximum(c0r[...] + c1r[...], 1.0)
        hv = jnp.dot(hr[...], rtr[...], preferred_element_type=_f32)
        o_ref[...] = jnp.maximum(agg + hv + kbr[...], 0.0)

    full = lambda s: pl.BlockSpec(s, lambda i: tuple(0 for _ in s))
    blk = pl.BlockSpec((b, _W), lambda i: (i, 0))
    return pl.pallas_call(
        body,
        grid=(_N // b,),
        in_specs=[blk, blk, blk, blk, blk, full((_W, _W)), full((1, _W))],
        out_specs=pl.BlockSpec((b, _W), lambda i: (i, 0)),
        out_shape=jax.ShapeDtypeStruct((_N, _W), _f32),
    )(p0, p1, c0, c1, h, root, kb)


def _tc_update_out(p0, p1, c0, c1, h, root, kb, wout, bout):
    b = 2000

    def body(p0r, p1r, c0r, c1r, hr, rtr, kbr, wor, bor, o_ref):
        agg = (p0r[...] + p1r[...]) / jnp.maximum(c0r[...] + c1r[...], 1.0)
        hv = jnp.dot(hr[...], rtr[...], preferred_element_type=_f32)
        hn = jnp.maximum(agg + hv + kbr[...], 0.0)
        o_ref[...] = (jnp.dot(hn, wor[...], preferred_element_type=_f32)
                      + bor[...])

    full = lambda s: pl.BlockSpec(s, lambda i: tuple(0 for _ in s))
    blk = pl.BlockSpec((b, _W), lambda i: (i, 0))
    return pl.pallas_call(
        body,
        grid=(_N // b,),
        in_specs=[blk, blk, blk, blk, blk, full((_W, _W)), full((1, _W)),
                  full((_W, 128)), full((1, 128))],
        out_specs=pl.BlockSpec((b, 128), lambda i: (i, 0)),
        out_shape=jax.ShapeDtypeStruct((_N, 128), _f32),
    )(p0, p1, c0, c1, h, root, kb, wout, bout)


@functools.cache
def _sc_kernels():
    mesh = plsc.VectorSubcoreMesh(core_axis_name="c", subcore_axis_name="s")
    params = pltpu.CompilerParams(use_tc_tiling_on_sc=False)

    @functools.partial(
        pl.kernel,
        out_type=[jax.ShapeDtypeStruct((_E_PAD, _W), _f32),
                  jax.ShapeDtypeStruct((_E_PAD, _W), _f32)],
        mesh=mesh,
        compiler_params=params,
        scratch_types=[pltpu.VMEM((_NCHUNK, _CH), jnp.int32),
                       pltpu.VMEM((_EW, _W), _f32),
                       pltpu.SemaphoreType.DMA],
    )
    def sc_gather(h_hbm, src_hbm, dst_hbm, xj_hbm, xi_hbm, idx_v, rows_v,
                  sem):
        w = lax.axis_index("s") * _NC + lax.axis_index("c")
        for idx_hbm, out_hbm in ((src_hbm, xj_hbm), (dst_hbm, xi_hbm)):
            pltpu.sync_copy(idx_hbm.at[w], idx_v)

            def fire(j, carry):
                pltpu.async_copy(h_hbm.at[idx_v.at[j]],
                                 rows_v.at[pl.ds(j * _CH, _CH)], sem)
                return carry

            def drain(j, carry):
                pltpu.make_async_copy(
                    h_hbm.at[idx_v.at[j]],
                    rows_v.at[pl.ds(j * _CH, _CH)], sem).wait()
                return carry

            lax.fori_loop(0, _NCHUNK, fire, 0)
            lax.fori_loop(0, _NCHUNK, drain, 0)
            pltpu.sync_copy(rows_v, out_hbm.at[pl.ds(w * _EW, _EW)])

    @functools.partial(
        pl.kernel,
        out_type=jax.ShapeDtypeStruct((_NC * _ND, _W), _f32),
        mesh=mesh,
        compiler_params=params,
        scratch_types=[pltpu.VMEM((_NCHUNK, _CH), jnp.int32),
                       pltpu.VMEM((_EW, _W), _f32),
                       pltpu.VMEM_SHARED((_ND, _W), _f32),
                       pltpu.SemaphoreType.DMA],
    )
    def sc_scatter(msg_hbm, dst_hbm, zeros_hbm, out_hbm, idx_v, msg_v,
                   agg_sh, sem):
        cid = lax.axis_index("c")
        sid = lax.axis_index("s")
        w = sid * _NC + cid
        pltpu.sync_copy(zeros_hbm.at[pl.ds(sid * _STRIPE, _STRIPE)],
                        agg_sh.at[pl.ds(sid * _STRIPE, _STRIPE)])
        plsc.subcore_barrier()
        pltpu.sync_copy(dst_hbm.at[w], idx_v)
        pltpu.sync_copy(msg_hbm.at[pl.ds(w * _EW, _EW)], msg_v)

        def fire(j, carry):
            pltpu.async_copy(msg_v.at[pl.ds(j * _CH, _CH)],
                             agg_sh.at[idx_v.at[j]], sem, add=True)
            return carry

        def drain(j, carry):
            pltpu.make_async_copy(msg_v.at[pl.ds(j * _CH, _CH)],
                                  agg_sh.at[idx_v.at[j]], sem).wait()
            return carry

        lax.fori_loop(0, _NCHUNK, fire, 0)
        lax.fori_loop(0, _NCHUNK, drain, 0)
        plsc.subcore_barrier()
        pltpu.sync_copy(agg_sh.at[pl.ds(sid * _STRIPE, _STRIPE)],
                        out_hbm.at[pl.ds(cid * _ND + sid * _STRIPE,
                                         _STRIPE)])

    @functools.partial(
        pl.kernel,
        out_type=jax.ShapeDtypeStruct((_NC * _ND, _W), _f32),
        mesh=mesh,
        compiler_params=params,
        scratch_types=[pltpu.VMEM((_NCHUNK, _CH), jnp.int32),
                       pltpu.VMEM((_CH, _W), _f32),
                       pltpu.VMEM_SHARED((_ND, _W), _f32),
                       pltpu.SemaphoreType.DMA],
    )
    def sc_count(dst_hbm, zeros_hbm, ones_hbm, out_hbm, idx_v, ones_v,
                 cnt_sh, sem):
        cid = lax.axis_index("c")
        sid = lax.axis_index("s")
        w = sid * _NC + cid
        pltpu.sync_copy(zeros_hbm.at[pl.ds(sid * _STRIPE, _STRIPE)],
                        cnt_sh.at[pl.ds(sid * _STRIPE, _STRIPE)])
        pltpu.sync_copy(ones_hbm, ones_v)
        plsc.subcore_barrier()
        pltpu.sync_copy(dst_hbm.at[w], idx_v)

        def fire(j, carry):
            pltpu.async_copy(ones_v, cnt_sh.at[idx_v.at[j]], sem, add=True)
            return carry

        def drain(j, carry):
            pltpu.make_async_copy(ones_v, cnt_sh.at[idx_v.at[j]], sem).wait()
            return carry

        lax.fori_loop(0, _NCHUNK, fire, 0)
        lax.fori_loop(0, _NCHUNK, drain, 0)
        plsc.subcore_barrier()
        pltpu.sync_copy(cnt_sh.at[pl.ds(sid * _STRIPE, _STRIPE)],
                        out_hbm.at[pl.ds(cid * _ND + sid * _STRIPE,
                                         _STRIPE)])

    return sc_gather, sc_scatter, sc_count


def _sc_gather(h, src3, dst3):
    return _sc_kernels()[0](h, src3, dst3)


def _sc_scatter(msg, dst3, zeros_nd):
    return _sc_kernels()[1](msg, dst3, zeros_nd)


def _sc_count(dst3, zeros_nd, ones_ch):
    return _sc_kernels()[2](dst3, zeros_nd, ones_ch)


def kernel(x, edge_index, edge_attr, W1, b1, Wout, bout, root_param, kbias,
           ps0_W, ps0_b, ps0_r, ps1_W, ps1_b, ps1_r, ps2_W, ps2_b, ps2_r,
           psout_W, psout_b, psout_r, bn_g, bn_b, dW1, db1, dW2, db2, dW3,
           db3):
    pad = _E_PAD - _E
    lidx = jnp.arange(_E_PAD, dtype=jnp.int32)
    blk = lidx // _BM
    loc = lidx % _BM
    perm = blk * _BM + (loc % 8) * (_BM // 8) + loc // 8
    src_pad = jnp.concatenate([edge_index[0], jnp.zeros((pad,), jnp.int32)])
    dst_pad = jnp.concatenate(
        [edge_index[1], jnp.full((pad,), _DUMMY, jnp.int32)])
    src3 = src_pad[perm].reshape(_NW, _NCHUNK, _CH)
    dst3 = dst_pad[perm].reshape(_NW, _NCHUNK, _CH)
    attr_dense = jnp.concatenate(
        [edge_attr.reshape(_E), jnp.zeros((pad,), _f32)]
    ).reshape(_E_PAD // 128, 128)
    zeros_nd = jnp.zeros((_ND, _W), _f32)
    ones_ch = jnp.ones((_CH, _W), _f32)

    eye = jnp.eye(_W, dtype=_f32)
    rep = jnp.kron(eye, jnp.ones((1, _W), _f32))
    red = jnp.kron(jnp.ones((_W, 1), _f32), eye)

    b1r = b1.reshape(1, _W)
    kbr = kbias.reshape(1, _W)
    boutr = bout.reshape(1, 128)
    bngr = bn_g.reshape(1, 64)
    bnbr = bn_b.reshape(1, 64)
    bf16 = jnp.bfloat16
    w0 = ps0_W.reshape(1, 256)
    b0 = ps0_b.reshape(1, 256)
    r0 = ps0_r.reshape(1, 256)
    w1c = jnp.transpose(ps1_W, (1, 0, 2)).reshape(64, 256).astype(bf16)
    b1c = ps1_b.reshape(1, 256)
    r1c = ps1_r.reshape(1, 256)
    w2c = jnp.transpose(ps2_W, (1, 0, 2)).reshape(64, 256).astype(bf16)
    b2c = ps2_b.reshape(1, 256)
    r2c = ps2_r.reshape(1, 256)
    woc = jnp.transpose(psout_W, (1, 0, 2)).reshape(64, 1024).astype(bf16)
    boc = psout_b.reshape(1, 1024)
    roc = psout_r.reshape(1, 1024)
    d1r = dW1.reshape(1, 128)
    e1r = db1.reshape(1, 128)
    e2r = db2.reshape(1, 128)
    e3r = db3.reshape(1, 256)
    dw2b = dW2.astype(bf16)
    dw3b = dW3.astype(bf16)

    h0 = _tc_h0(x, W1, b1r)
    cnts = _sc_count(dst3, zeros_nd, ones_ch)
    st1, kx1a = _tc_stats1(attr_dense, w0, b0, r0, w1c, b1c, r1c)
    st2, kx2a = _tc_stats2(kx1a, st1, bngr, bnbr, w2c, b2c, r2c)
    wk, wsum = _tc_weights(attr_dense, kx2a, bngr, bnbr, st2,
                           woc, boc, roc,
                           d1r, e1r, dw2b, e2r, dw3b, e3r)
    c0 = cnts[:_N]
    c1 = cnts[_ND:_ND + _N]

    h = h0
    for layer in range(2):
        xj, xi = _sc_gather(h, src3, dst3)
        msg = _tc_msg(xj.reshape(_EP, 128), xi.reshape(_EP, 128),
                      wk, wsum, rep, red)
        parts = _sc_scatter(msg.reshape(_E_PAD, _W), dst3, zeros_nd)
        p0 = parts[:_N]
        p1 = parts[_ND:_ND + _N]
        if layer == 0:
            h = _tc_update(p0, p1, c0, c1, h, root_param, kbr)
        else:
            h = _tc_update_out(p0, p1, c0, c1, h, root_param, kbr, Wout,
                               boutr)
    return h


# speedup vs baseline: 4.8475x; 1.0756x over previous
import functools

import jax
import jax.numpy as jnp
from jax import lax
from jax.experimental import pallas as pl
from jax.experimental.pallas import tpu as pltpu
from jax.experimental.pallas import tpu_sc as plsc

_N = 10000
_E = 160000
_W = 16
_NC = 2
_NS = 16
_NW = _NC * _NS
_CH = 128
_NCHUNK = 40
_EW = _CH * _NCHUNK
_E_PAD = _EW * _NW
_EP = _E_PAD // 8
_ND = 10240
_STRIPE = _ND // _NS
_DUMMY = _N
_BM = 2048
_BMSG = 4096

_f32 = jnp.float32


def _leaky(x):
    return jnp.where(x >= 0, x, 0.1 * x)


def _ps_combine(y, r_ref, k):
    y0 = y[:, 0:k]
    y1 = _leaky(y[:, k:2 * k])
    y2 = _leaky(y[:, 2 * k:3 * k])
    y3 = _leaky(y[:, 3 * k:4 * k])
    return (r_ref[:, 0:k] * y0 + r_ref[:, k:2 * k] * y1
            + r_ref[:, 2 * k:3 * k] * (y2 * y2)
            + r_ref[:, 3 * k:4 * k] * (y3 * y3 * y3))


def _ps_scalar(a, w_ref, b_ref, r_ref):
    y = a * w_ref[...] + b_ref[...]
    return _ps_combine(y, r_ref, 64)


def _ps_mat(x, w_ref, b_ref, r_ref, k):
    y = jnp.dot(x.astype(jnp.bfloat16), w_ref[...],
                preferred_element_type=_f32) + b_ref[...]
    return _ps_combine(y, r_ref, k)


def _attr_col(a_ref, b):
    rows = b // 128
    eye = jnp.eye(rows, dtype=_f32)
    at = lax.dot_general(a_ref[...], eye, (((0,), (0,)), ((), ())),
                         precision=lax.Precision.HIGHEST,
                         preferred_element_type=_f32)
    return jnp.concatenate(
        [at[:, r:r + 1] for r in range(rows)], axis=0)


def _ps_mat_bf16(x, w_ref, b_ref, r_ref, k):
    bf = jnp.bfloat16
    y = (jnp.dot(x.astype(bf), w_ref[...], preferred_element_type=_f32)
         + b_ref[...]).astype(bf)
    r = r_ref[...].astype(bf)
    y0 = y[:, 0:k]
    y1 = _leaky(y[:, k:2 * k])
    y2 = _leaky(y[:, 2 * k:3 * k])
    y3 = _leaky(y[:, 3 * k:4 * k])
    return (r[:, 0:k] * y0 + r[:, k:2 * k] * y1
            + r[:, 2 * k:3 * k] * (y2 * y2)
            + r[:, 3 * k:4 * k] * (y3 * y3 * y3))


def _bn_apply(x, st_ref, g_ref, b_ref):
    return ((x - st_ref[0:1, :]) * lax.rsqrt(st_ref[1:2, :] + 1e-5)
            * g_ref[0:1, :] + b_ref[0:1, :])


def _tc_h0(x, w1, b1):
    b = 2000

    def body(x_ref, w_ref, b_ref, o_ref):
        o_ref[...] = (jnp.dot(x_ref[...], w_ref[...],
                              preferred_element_type=_f32) + b_ref[...])

    return pl.pallas_call(
        body,
        grid=(_N // b,),
        in_specs=[pl.BlockSpec((b, 128), lambda i: (i, 0)),
                  pl.BlockSpec((128, _W), lambda i: (0, 0)),
                  pl.BlockSpec((1, _W), lambda i: (0, 0))],
        out_specs=pl.BlockSpec((b, _W), lambda i: (i, 0)),
        out_shape=jax.ShapeDtypeStruct((_N, _W), _f32),
    )(x, w1, b1)


def _stats_finalize(o_ref, nblk, i):
    @pl.when(i == nblk - 1)
    def _():
        s = o_ref[0:1, :]
        q = o_ref[1:2, :]
        m = s * (1.0 / _E)
        v = q * (1.0 / _E) - m * m
        o_ref[0:1, :] = m
        o_ref[1:2, :] = v


def _tc_stats1(attr, w0, b0, r0, w1, b1, r1):
    b = 2048
    nblk = _E_PAD // b

    def body(a_ref, w0r, b0r, r0r, w1r, b1r, r1r, o_ref, k_ref):
        i = pl.program_id(0)

        @pl.when(i == 0)
        def _():
            o_ref[...] = jnp.zeros((2, 64), _f32)

        kx0 = _ps_scalar(_attr_col(a_ref, b), w0r, b0r, r0r)
        kx1 = _ps_mat(kx0, w1r, b1r, r1r, 64)
        k_ref[...] = kx1
        rows = lax.broadcasted_iota(jnp.int32, (b, 64), 0)
        kx1 = jnp.where(rows < _E - i * b, kx1, 0.0)
        o_ref[0:1, :] += jnp.sum(kx1, axis=0, keepdims=True)
        o_ref[1:2, :] += jnp.sum(kx1 * kx1, axis=0, keepdims=True)
        _stats_finalize(o_ref, nblk, i)

    full = lambda s: pl.BlockSpec(s, lambda i: tuple(0 for _ in s))
    return pl.pallas_call(
        body,
        grid=(nblk,),
        in_specs=[pl.BlockSpec((b // 128, 128), lambda i: (i, 0)),
                  full((1, 256)), full((1, 256)), full((1, 256)),
                  full((64, 256)), full((1, 256)), full((1, 256))],
        out_specs=[pl.BlockSpec((2, 64), lambda i: (0, 0)),
                   pl.BlockSpec((b, 64), lambda i: (i, 0))],
        out_shape=[jax.ShapeDtypeStruct((2, 64), _f32),
                   jax.ShapeDtypeStruct((_E_PAD, 64), _f32)],
    )(attr, w0, b0, r0, w1, b1, r1)


def _tc_stats2(kx1a, st1, bng, bnb, w2, b2, r2):
    b = 2048
    nblk = _E_PAD // b

    def body(k1_ref, st1r, gr, br, w2r, b2r, r2r, o_ref, k_ref):
        i = pl.program_id(0)

        @pl.when(i == 0)
        def _():
            o_ref[...] = jnp.zeros((2, 64), _f32)

        bn1 = _bn_apply(k1_ref[...], st1r, gr, br)
        kx2 = _ps_mat(bn1, w2r, b2r, r2r, 64)
        k_ref[...] = kx2
        rows = lax.broadcasted_iota(jnp.int32, (b, 64), 0)
        kx2 = jnp.where(rows < _E - i * b, kx2, 0.0)
        o_ref[0:1, :] += jnp.sum(kx2, axis=0, keepdims=True)
        o_ref[1:2, :] += jnp.sum(kx2 * kx2, axis=0, keepdims=True)
        _stats_finalize(o_ref, nblk, i)

    full = lambda s: pl.BlockSpec(s, lambda i: tuple(0 for _ in s))
    return pl.pallas_call(
        body,
        grid=(nblk,),
        in_specs=[pl.BlockSpec((b, 64), lambda i: (i, 0)),
                  full((2, 64)), full((1, 64)), full((1, 64)),
                  full((64, 256)), full((1, 256)), full((1, 256))],
        out_specs=[pl.BlockSpec((2, 64), lambda i: (0, 0)),
                   pl.BlockSpec((b, 64), lambda i: (i, 0))],
        out_shape=[jax.ShapeDtypeStruct((2, 64), _f32),
                   jax.ShapeDtypeStruct((_E_PAD, 64), _f32)],
    )(kx1a, st1, bng, bnb, w2, b2, r2)


def _tc_weights(attr_pad, kx2a, bng, bnb,
                st2, wo, bo, ro, dw1, db1, dw2, db2, dw3, db3):
    b = _BM
    nblk = _E_PAD // b

    def body(a_ref, k2_ref, gr, br,
             st2r, wor, bor, ror, d1r, e1r, d2r, e2r, d3r, e3r,
             wk_ref, ws_ref):
        a = _attr_col(a_ref, b)
        bn2 = _bn_apply(k2_ref[...], st2r, gr, br)
        wk = _ps_mat_bf16(bn2, wor, bor, ror, 256)
        hd = jnp.maximum(a * d1r[...] + e1r[...], 0.0)
        hd = jnp.maximum(
            jnp.dot(hd.astype(jnp.bfloat16), d2r[...],
                    preferred_element_type=_f32) + e2r[...], 0.0)
        wop = jnp.dot(hd.astype(jnp.bfloat16), d3r[...],
                      preferred_element_type=_f32) + e3r[...]
        wk_ref[...] = wk
        ws_ref[...] = (wk.astype(_f32) + wop).astype(jnp.bfloat16)

    full = lambda s: pl.BlockSpec(s, lambda i: tuple(0 for _ in s))
    return pl.pallas_call(
        body,
        grid=(nblk,),
        in_specs=[pl.BlockSpec((b // 128, 128), lambda i: (i, 0)),
                  pl.BlockSpec((b, 64), lambda i: (i, 0)),
                  full((1, 64)), full((1, 64)),
                  full((2, 64)),
                  full((64, 1024)), full((1, 1024)), full((1, 1024)),
                  full((1, 128)), full((1, 128)),
                  full((128, 128)), full((1, 128)),
                  full((128, 256)), full((1, 256))],
        out_specs=[pl.BlockSpec((b, 256), lambda i: (i, 0)),
                   pl.BlockSpec((b, 256), lambda i: (i, 0))],
        out_shape=[jax.ShapeDtypeStruct((_E_PAD, 256), jnp.bfloat16),
                   jax.ShapeDtypeStruct((_E_PAD, 256), jnp.bfloat16)],
    )(attr_pad, kx2a, bng, bnb,
      st2, wo, bo, ro, dw1, db1, dw2, db2, dw3, db3)


def _tc_msg(xjp, xip, wk, wsum, rep, red):
    b = _BMSG
    bp = b // 8
    nblk = _E_PAD // b

    def body(xj_ref, xi_ref, wk_ref, ws_ref, rep_ref, red_ref, o_ref):
        xp = xj_ref[...]
        ip = xi_ref[...]
        xjv = jnp.concatenate(
            [xp[:, 16 * p:16 * (p + 1)] for p in range(8)], axis=0)
        xiv = jnp.concatenate(
            [ip[:, 16 * p:16 * (p + 1)] for p in range(8)], axis=0)
        xjr = jnp.dot(xjv, rep_ref[...], preferred_element_type=_f32)
        xir = jnp.dot(xiv, rep_ref[...], preferred_element_type=_f32)
        t = (xjr * ws_ref[...].astype(_f32)
             - xir * wk_ref[...].astype(_f32))
        msg = jnp.dot(t, red_ref[...], preferred_element_type=_f32)
        for p in range(8):
            o_ref[:, 16 * p:16 * (p + 1)] = msg[bp * p:bp * (p + 1), :]

    full = lambda s: pl.BlockSpec(s, lambda i: tuple(0 for _ in s))
    return pl.pallas_call(
        body,
        grid=(nblk,),
        in_specs=[pl.BlockSpec((bp, 128), lambda i: (i, 0)),
                  pl.BlockSpec((bp, 128), lambda i: (i, 0)),
                  pl.BlockSpec((b, 256), lambda i: (i, 0)),
                  pl.BlockSpec((b, 256), lambda i: (i, 0)),
                  full((_W, 256)), full((256, _W))],
        out_specs=pl.BlockSpec((bp, 128), lambda i: (i, 0)),
        out_shape=jax.ShapeDtypeStruct((_EP, 128), _f32),
    )(xjp, xip, wk, wsum, rep, red)


def _tc_update(p0, p1, c0, c1, h, root, kb):
    b = 2000

    def body(p0r, p1r, c0r, c1r, hr, rtr, kbr, o_ref):
        agg = (p0r[...] + p1r[...]) / jnp.maximum(c0r[...] + c1r[...], 1.0)
        hv = jnp.dot(hr[...], rtr[...], preferred_element_type=_f32)
        o_ref[...] = jnp.maximum(agg + hv + kbr[...], 0.0)

    full = lambda s: pl.BlockSpec(s, lambda i: tuple(0 for _ in s))
    blk = pl.BlockSpec((b, _W), lambda i: (i, 0))
    return pl.pallas_call(
        body,
        grid=(_N // b,),
        in_specs=[blk, blk, blk, blk, blk, full((_W, _W)), full((1, _W))],
        out_specs=pl.BlockSpec((b, _W), lambda i: (i, 0)),
        out_shape=jax.ShapeDtypeStruct((_N, _W), _f32),
    )(p0, p1, c0, c1, h, root, kb)


def _tc_update_out(p0, p1, c0, c1, h, root, kb, wout, bout):
    b = 2000

    def body(p0r, p1r, c0r, c1r, hr, rtr, kbr, wor, bor, o_ref):
        agg = (p0r[...] + p1r[...]) / jnp.maximum(c0r[...] + c1r[...], 1.0)
        hv = jnp.dot(hr[...], rtr[...], preferred_element_type=_f32)
        hn = jnp.maximum(agg + hv + kbr[...], 0.0)
        o_ref[...] = (jnp.dot(hn, wor[...], preferred_element_type=_f32)
                      + bor[...])

    full = lambda s: pl.BlockSpec(s, lambda i: tuple(0 for _ in s))
    blk = pl.BlockSpec((b, _W), lambda i: (i, 0))
    return pl.pallas_call(
        body,
        grid=(_N // b,),
        in_specs=[blk, blk, blk, blk, blk, full((_W, _W)), full((1, _W)),
                  full((_W, 128)), full((1, 128))],
        out_specs=pl.BlockSpec((b, 128), lambda i: (i, 0)),
        out_shape=jax.ShapeDtypeStruct((_N, 128), _f32),
    )(p0, p1, c0, c1, h, root, kb, wout, bout)


@functools.cache
def _sc_kernels():
    mesh = plsc.VectorSubcoreMesh(core_axis_name="c", subcore_axis_name="s")
    params = pltpu.CompilerParams(use_tc_tiling_on_sc=False)

    @functools.partial(
        pl.kernel,
        out_type=[jax.ShapeDtypeStruct((_E_PAD, _W), _f32),
                  jax.ShapeDtypeStruct((_E_PAD, _W), _f32)],
        mesh=mesh,
        compiler_params=params,
        scratch_types=[pltpu.VMEM((_NCHUNK, _CH), jnp.int32),
                       pltpu.VMEM((_EW, _W), _f32),
                       pltpu.SemaphoreType.DMA],
    )
    def sc_gather(h_hbm, src_hbm, dst_hbm, xj_hbm, xi_hbm, idx_v, rows_v,
                  sem):
        w = lax.axis_index("s") * _NC + lax.axis_index("c")
        for idx_hbm, out_hbm in ((src_hbm, xj_hbm), (dst_hbm, xi_hbm)):
            pltpu.sync_copy(idx_hbm.at[w], idx_v)

            def fire(j, carry):
                pltpu.async_copy(h_hbm.at[idx_v.at[j]],
                                 rows_v.at[pl.ds(j * _CH, _CH)], sem)
                return carry

            def drain(j, carry):
                pltpu.make_async_copy(
                    h_hbm.at[idx_v.at[j]],
                    rows_v.at[pl.ds(j * _CH, _CH)], sem).wait()
                return carry

            lax.fori_loop(0, _NCHUNK, fire, 0)
            lax.fori_loop(0, _NCHUNK, drain, 0)
            pltpu.sync_copy(rows_v, out_hbm.at[pl.ds(w * _EW, _EW)])

    @functools.partial(
        pl.kernel,
        out_type=jax.ShapeDtypeStruct((_NC * _ND, _W), _f32),
        mesh=mesh,
        compiler_params=params,
        scratch_types=[pltpu.VMEM((_NCHUNK, _CH), jnp.int32),
                       pltpu.VMEM((_EW, _W), _f32),
                       pltpu.VMEM_SHARED((_ND, _W), _f32),
                       pltpu.SemaphoreType.DMA],
    )
    def sc_scatter(msg_hbm, dst_hbm, zeros_hbm, out_hbm, idx_v, msg_v,
                   agg_sh, sem):
        cid = lax.axis_index("c")
        sid = lax.axis_index("s")
        w = sid * _NC + cid
        pltpu.sync_copy(zeros_hbm.at[pl.ds(sid * _STRIPE, _STRIPE)],
                        agg_sh.at[pl.ds(sid * _STRIPE, _STRIPE)])
        plsc.subcore_barrier()
        pltpu.sync_copy(dst_hbm.at[w], idx_v)
        pltpu.sync_copy(msg_hbm.at[pl.ds(w * _EW, _EW)], msg_v)

        def fire(j, carry):
            pltpu.async_copy(msg_v.at[pl.ds(j * _CH, _CH)],
                             agg_sh.at[idx_v.at[j]], sem, add=True)
            return carry

        def drain(j, carry):
            pltpu.make_async_copy(msg_v.at[pl.ds(j * _CH, _CH)],
                                  agg_sh.at[idx_v.at[j]], sem).wait()
            return carry

        lax.fori_loop(0, _NCHUNK, fire, 0)
        lax.fori_loop(0, _NCHUNK, drain, 0)
        plsc.subcore_barrier()
        pltpu.sync_copy(agg_sh.at[pl.ds(sid * _STRIPE, _STRIPE)],
                        out_hbm.at[pl.ds(cid * _ND + sid * _STRIPE,
                                         _STRIPE)])

    @functools.partial(
        pl.kernel,
        out_type=jax.ShapeDtypeStruct((_NC * _ND, _W), _f32),
        mesh=mesh,
        compiler_params=params,
        scratch_types=[pltpu.VMEM((_NCHUNK, _CH), jnp.int32),
                       pltpu.VMEM((_CH, _W), _f32),
                       pltpu.VMEM_SHARED((_ND, _W), _f32),
                       pltpu.SemaphoreType.DMA],
    )
    def sc_count(dst_hbm, zeros_hbm, ones_hbm, out_hbm, idx_v, ones_v,
                 cnt_sh, sem):
        cid = lax.axis_index("c")
        sid = lax.axis_index("s")
        w = sid * _NC + cid
        pltpu.sync_copy(zeros_hbm.at[pl.ds(sid * _STRIPE, _STRIPE)],
                        cnt_sh.at[pl.ds(sid * _STRIPE, _STRIPE)])
        pltpu.sync_copy(ones_hbm, ones_v)
        plsc.subcore_barrier()
        pltpu.sync_copy(dst_hbm.at[w], idx_v)

        def fire(j, carry):
            pltpu.async_copy(ones_v, cnt_sh.at[idx_v.at[j]], sem, add=True)
            return carry

        def drain(j, carry):
            pltpu.make_async_copy(ones_v, cnt_sh.at[idx_v.at[j]], sem).wait()
            return carry

        lax.fori_loop(0, _NCHUNK, fire, 0)
        lax.fori_loop(0, _NCHUNK, drain, 0)
        plsc.subcore_barrier()
        pltpu.sync_copy(cnt_sh.at[pl.ds(sid * _STRIPE, _STRIPE)],
                        out_hbm.at[pl.ds(cid * _ND + sid * _STRIPE,
                                         _STRIPE)])

    return sc_gather, sc_scatter, sc_count


def _sc_gather(h, src3, dst3):
    return _sc_kernels()[0](h, src3, dst3)


def _sc_scatter(msg, dst3, zeros_nd):
    return _sc_kernels()[1](msg, dst3, zeros_nd)


def _sc_count(dst3, zeros_nd, ones_ch):
    return _sc_kernels()[2](dst3, zeros_nd, ones_ch)


def kernel(x, edge_index, edge_attr, W1, b1, Wout, bout, root_param, kbias,
           ps0_W, ps0_b, ps0_r, ps1_W, ps1_b, ps1_r, ps2_W, ps2_b, ps2_r,
           psout_W, psout_b, psout_r, bn_g, bn_b, dW1, db1, dW2, db2, dW3,
           db3):
    pad = _E_PAD - _E
    lidx = jnp.arange(_E_PAD, dtype=jnp.int32)
    blk = lidx // _BMSG
    loc = lidx % _BMSG
    perm = blk * _BMSG + (loc % 8) * (_BMSG // 8) + loc // 8
    src_pad = jnp.concatenate([edge_index[0], jnp.zeros((pad,), jnp.int32)])
    dst_pad = jnp.concatenate(
        [edge_index[1], jnp.full((pad,), _DUMMY, jnp.int32)])
    src3 = src_pad[perm].reshape(_NW, _NCHUNK, _CH)
    dst3 = dst_pad[perm].reshape(_NW, _NCHUNK, _CH)
    attr_dense = jnp.concatenate(
        [edge_attr.reshape(_E), jnp.zeros((pad,), _f32)]
    ).reshape(_E_PAD // 128, 128)
    zeros_nd = jnp.zeros((_ND, _W), _f32)
    ones_ch = jnp.ones((_CH, _W), _f32)

    eye = jnp.eye(_W, dtype=_f32)
    rep = jnp.kron(eye, jnp.ones((1, _W), _f32))
    red = jnp.kron(jnp.ones((_W, 1), _f32), eye)

    b1r = b1.reshape(1, _W)
    kbr = kbias.reshape(1, _W)
    boutr = bout.reshape(1, 128)
    bngr = bn_g.reshape(1, 64)
    bnbr = bn_b.reshape(1, 64)
    bf16 = jnp.bfloat16
    w0 = ps0_W.reshape(1, 256)
    b0 = ps0_b.reshape(1, 256)
    r0 = ps0_r.reshape(1, 256)
    w1c = jnp.transpose(ps1_W, (1, 0, 2)).reshape(64, 256).astype(bf16)
    b1c = ps1_b.reshape(1, 256)
    r1c = ps1_r.reshape(1, 256)
    w2c = jnp.transpose(ps2_W, (1, 0, 2)).reshape(64, 256).astype(bf16)
    b2c = ps2_b.reshape(1, 256)
    r2c = ps2_r.reshape(1, 256)
    woc = jnp.transpose(psout_W, (1, 0, 2)).reshape(64, 1024).astype(bf16)
    boc = psout_b.reshape(1, 1024)
    roc = psout_r.reshape(1, 1024)
    d1r = dW1.reshape(1, 128)
    e1r = db1.reshape(1, 128)
    e2r = db2.reshape(1, 128)
    e3r = db3.reshape(1, 256)
    dw2b = dW2.astype(bf16)
    dw3b = dW3.astype(bf16)

    h0 = _tc_h0(x, W1, b1r)
    cnts = _sc_count(dst3, zeros_nd, ones_ch)
    st1, kx1a = _tc_stats1(attr_dense, w0, b0, r0, w1c, b1c, r1c)
    st2, kx2a = _tc_stats2(kx1a, st1, bngr, bnbr, w2c, b2c, r2c)
    wk, wsum = _tc_weights(attr_dense, kx2a, bngr, bnbr, st2,
                           woc, boc, roc,
                           d1r, e1r, dw2b, e2r, dw3b, e3r)
    c0 = cnts[:_N]
    c1 = cnts[_ND:_ND + _N]

    h = h0
    for layer in range(2):
        xj, xi = _sc_gather(h, src3, dst3)
        msg = _tc_msg(xj.reshape(_EP, 128), xi.reshape(_EP, 128),
                      wk, wsum, rep, red)
        parts = _sc_scatter(msg.reshape(_E_PAD, _W), dst3, zeros_nd)
        p0 = parts[:_N]
        p1 = parts[_ND:_ND + _N]
        if layer == 0:
            h = _tc_update(p0, p1, c0, c1, h, root_param, kbr)
        else:
            h = _tc_update_out(p0, p1, c0, c1, h, root_param, kbr, Wout,
                               boutr)
    return h
